# baseline (speedup 1.0000x reference)
import functools

import jax
import jax.numpy as jnp
from jax import lax
from jax.experimental import pallas as pl
from jax.experimental.pallas import tpu as pltpu
from jax.experimental.pallas import tpu_sc as plsc

F32 = jnp.float32
I32 = jnp.int32
INV_SQRT2 = 0.7071067811865475
ACT_SCALE = 1.0 / 0.6

N_ATOMS = 10000
N_EDGES = 160000
N_TRIP = 320000


def _act(x):
    return jax.nn.silu(x) * ACT_SCALE


def _res(x, w0, w1):
    y = _act(x @ w0[...])
    y = _act(y @ w1[...])
    return (x + y) * INV_SQRT2


def _edge_pre_kernel(m_ref, rbf3_ref, wba, wrbf, wdown, xba_ref):
    t = _act(m_ref[...] @ wba[...])
    t = t * (rbf3_ref[...] @ wrbf[...])
    xba_ref[...] = _act(t @ wdown[...]).astype(jnp.bfloat16)


def _edge_pre(m, rbf3, W_ba, W_rbf, W_down):
    E, BE = N_EDGES, 2000
    full = lambda s: pl.BlockSpec(s, lambda i: (0, 0))
    return pl.pallas_call(
        _edge_pre_kernel,
        grid=(E // BE,),
        in_specs=[pl.BlockSpec((BE, 128), lambda i: (i, 0)),
                  pl.BlockSpec((BE, 16), lambda i: (i, 0)),
                  full((128, 128)), full((16, 128)), full((128, 64))],
        out_specs=pl.BlockSpec((BE, 64), lambda i: (i, 0)),
        out_shape=jax.ShapeDtypeStruct((E, 64), jnp.bfloat16),
    )(m, rbf3, W_ba, W_rbf, W_down)


def _bilinear_kernel(x_ref, c_ref, w_ref, out_ref, z_ref):
    x = x_ref[...]
    c = c_ref[...].astype(jnp.bfloat16)
    z = jnp.concatenate([x * c[:, k:k + 1] for k in range(16)], axis=1)
    out_ref[...] = jnp.dot(z, w_ref[...], preferred_element_type=F32)


def _bilinear(x_ba_t, cbf3, Wb_t):
    T, BT = N_TRIP, 2000
    return pl.pallas_call(
        _bilinear_kernel,
        grid=(T // BT,),
        in_specs=[pl.BlockSpec((BT, 64), lambda i: (i, 0)),
                  pl.BlockSpec((BT, 16), lambda i: (i, 0)),
                  pl.BlockSpec((1024, 64), lambda i: (0, 0)),
                  ],
        out_specs=pl.BlockSpec((BT, 64), lambda i: (i, 0)),
        out_shape=jax.ShapeDtypeStruct((T, 64), F32),
        scratch_shapes=[pltpu.VMEM((BT, 1024), jnp.bfloat16)],
    )(x_ba_t, cbf3, Wb_t)


def _edge_mid_kernel(m_ref, x_ref, xsw_ref, rbfh_ref,
                     wca, wupca, wupac, bef0, bef1, a00, a01, a10, a11,
                     wrbfh, m1_ref, xm_ref):
    x_ca = _act(x_ref[...] @ wupca[...])
    x_ac = _act(xsw_ref[...] @ wupac[...])
    x3 = (x_ca + x_ac) * INV_SQRT2
    x4 = (_act(m_ref[...] @ wca[...]) + x3) * INV_SQRT2
    x4 = _res(x4, bef0, bef1)
    m1 = (m_ref[...] + x4) * INV_SQRT2
    m1 = _res(m1, a00, a01)
    m1 = _res(m1, a10, a11)
    m1_ref[...] = m1
    xm_ref[...] = m1 * (rbfh_ref[...] @ wrbfh[...])


def _edge_mid(m, x, x_sw, rbf_h, wca, wupca, wupac, bef0, bef1,
              a00, a01, a10, a11, wrbfh):
    E, BE = N_EDGES, 2000
    full = lambda s: pl.BlockSpec(s, lambda i: tuple(0 for _ in s))
    return pl.pallas_call(
        _edge_mid_kernel,
        grid=(E // BE,),
        in_specs=[pl.BlockSpec((BE, 128), lambda i: (i, 0)),
                  pl.BlockSpec((BE, 64), lambda i: (i, 0)),
                  pl.BlockSpec((BE, 64), lambda i: (i, 0)),
                  pl.BlockSpec((BE, 16), lambda i: (i, 0)),
                  full((128, 128)), full((64, 128)), full((64, 128)),
                  full((128, 128)), full((128, 128)), full((128, 128)),
                  full((128, 128)), full((128, 128)), full((128, 128)),
                  full((16, 128))],
        out_specs=[pl.BlockSpec((BE, 128), lambda i: (i, 0)),
                   pl.BlockSpec((BE, 128), lambda i: (i, 0))],
        out_shape=[jax.ShapeDtypeStruct((E, 128), F32),
                   jax.ShapeDtypeStruct((E, 128), F32)],
    )(m, x, x_sw, rbf_h, wca, wupca, wupac, bef0, bef1, a00, a01, a10,
      a11, wrbfh)


def _atom_kernel(x2_ref, h_ref, w1, r00, r01, r10, r11, r20, r21,
                 wc1, wc2, hout_ref, g1_ref, g2_ref):
    xh = _act(x2_ref[...] @ w1[...])
    xh = _res(xh, r00, r01)
    xh = _res(xh, r10, r11)
    xh = _res(xh, r20, r21)
    hn = (h_ref[...] + xh) * INV_SQRT2
    hout_ref[...] = hn
    g1_ref[...] = (hn @ wc1[...]).astype(jnp.bfloat16)
    g2_ref[...] = (hn @ wc2[...]).astype(jnp.bfloat16)


def _atoms(x2, h, w1, r00, r01, r10, r11, r20, r21, wc1, wc2):
    A, BA = N_ATOMS, 2000
    full = lambda s: pl.BlockSpec(s, lambda i: tuple(0 for _ in s))
    return pl.pallas_call(
        _atom_kernel,
        grid=(A // BA,),
        in_specs=[pl.BlockSpec((BA, 128), lambda i: (i, 0)),
                  pl.BlockSpec((BA, 128), lambda i: (i, 0)),
                  full((128, 128)), full((128, 128)), full((128, 128)),
                  full((128, 128)), full((128, 128)), full((128, 128)),
                  full((128, 128)), full((128, 128)), full((128, 128))],
        out_specs=[pl.BlockSpec((BA, 128), lambda i: (i, 0))] * 3,
        out_shape=[jax.ShapeDtypeStruct((A, 128), F32),
                   jax.ShapeDtypeStruct((A, 128), jnp.bfloat16),
                   jax.ShapeDtypeStruct((A, 128), jnp.bfloat16)],
    )(x2, h, w1, r00, r01, r10, r11, r20, r21, wc1, wc2)


def _edge_fin_kernel(g1_ref, g2_ref, m1_ref, wc3, rm0, rm1, out_ref):
    m1 = m1_ref[...]
    m2 = _act(g1_ref[...].astype(F32) + g2_ref[...].astype(F32)
              + m1 @ wc3[...])
    m2 = _res(m2, rm0, rm1)
    out_ref[...] = (m1 + m2) * INV_SQRT2


def _edge_fin(g1c, g2a, m1, wc3, rm0, rm1):
    E, BE = N_EDGES, 2000
    full = lambda s: pl.BlockSpec(s, lambda i: (0, 0))
    return pl.pallas_call(
        _edge_fin_kernel,
        grid=(E // BE,),
        in_specs=[pl.BlockSpec((BE, 128), lambda i: (i, 0)),
                  pl.BlockSpec((BE, 128), lambda i: (i, 0)),
                  pl.BlockSpec((BE, 128), lambda i: (i, 0)),
                  full((128, 128)), full((128, 128)), full((128, 128))],
        out_specs=pl.BlockSpec((BE, 128), lambda i: (i, 0)),
        out_shape=jax.ShapeDtypeStruct((E, 128), F32),
    )(g1c, g2a, m1, wc3, rm0, rm1)


_MESH = dict(core_axis_name="c", subcore_axis_name="s")
_SC_PARAMS = pltpu.CompilerParams(use_tc_tiling_on_sc=False,
                                  needs_layout_passes=False)


def _pick_nb(n):
    return next(nb for nb in (6, 4, 3, 2, 1) if n % nb == 0)


def _sc_gather(src, idx):
    N, W = src.shape
    dt = src.dtype
    B = idx.shape[0]
    per = B // 32
    CH = 128
    nfull = per // CH
    tail = per - nfull * CH
    NB = _pick_nb(nfull)
    NG = nfull // NB
    scratch = [pltpu.VMEM((per,), I32), pltpu.VMEM((NB * CH, W), dt),
               pltpu.SemaphoreType.DMA, pltpu.SemaphoreType.DMA]
    if tail:
        scratch += [pltpu.VMEM((tail, W), dt)]

    @functools.partial(
        pl.kernel, out_type=jax.ShapeDtypeStruct((B, W), dt),
        mesh=plsc.VectorSubcoreMesh(**_MESH), scratch_types=scratch,
        compiler_params=_SC_PARAMS)
    def k(src_hbm, idx_hbm, out_hbm, ids_v, rows_v, semg, semo, *rest):
        wid = lax.axis_index("s") * 2 + lax.axis_index("c")
        base = wid * per
        pltpu.sync_copy(idx_hbm.at[pl.ds(base, per)], ids_v)

        def group(g, carry):
            jo = g * NB * CH

            @pl.when(g > 0)
            def _():
                pltpu.make_async_copy(
                    rows_v, out_hbm.at[pl.ds(base, NB * CH)], semo).wait()

            descs = []
            for b in range(NB):
                descs.append(pltpu.async_copy(
                    src_hbm.at[ids_v.at[pl.ds(jo + b * CH, CH)]],
                    rows_v.at[pl.ds(b * CH, CH)], semg))
            for d in descs:
                d.wait()
            pltpu.async_copy(rows_v, out_hbm.at[pl.ds(base + jo, NB * CH)],
                             semo)
            return carry

        lax.fori_loop(0, NG, group, 0)
        if NG > 0:
            pltpu.make_async_copy(
                rows_v, out_hbm.at[pl.ds(base, NB * CH)], semo).wait()
        if tail:
            (rows_t,) = rest
            off = base + nfull * CH
            pltpu.async_copy(
                src_hbm.at[ids_v.at[pl.ds(nfull * CH, tail)]],
                rows_t, semg).wait()
            pltpu.sync_copy(rows_t, out_hbm.at[pl.ds(off, tail)])

    return k(src, idx)


def _sc_gather2(src1, idx_c, src2, idx_a):
    N, W = src1.shape
    dt = src1.dtype
    B = idx_c.shape[0]
    per = B // 32
    CH = 128
    nfull = per // CH
    tail = per - nfull * CH
    NB = _pick_nb(nfull)
    NG = nfull // NB
    out_t = [jax.ShapeDtypeStruct((B, W), dt)] * 2
    scratch = [pltpu.VMEM((per,), I32), pltpu.VMEM((per,), I32),
               pltpu.VMEM((NB * CH, W), dt), pltpu.VMEM((NB * CH, W), dt),
               pltpu.SemaphoreType.DMA, pltpu.SemaphoreType.DMA]
    if tail:
        scratch += [pltpu.VMEM((tail, W), dt), pltpu.VMEM((tail, W), dt)]

    @functools.partial(
        pl.kernel, out_type=out_t,
        mesh=plsc.VectorSubcoreMesh(**_MESH), scratch_types=scratch,
        compiler_params=_SC_PARAMS)
    def k(src_hbm, src2_hbm, idxc_hbm, idxa_hbm, outc_hbm, outa_hbm,
          idc_v, ida_v, rowsc_v, rowsa_v, semg, semo, *rest):
        wid = lax.axis_index("s") * 2 + lax.axis_index("c")
        base = wid * per
        pltpu.sync_copy(idxc_hbm.at[pl.ds(base, per)], idc_v)
        pltpu.sync_copy(idxa_hbm.at[pl.ds(base, per)], ida_v)

        def group(g, carry):
            jo = g * NB * CH

            @pl.when(g > 0)
            def _():
                pltpu.make_async_copy(
                    rowsc_v, outc_hbm.at[pl.ds(base, NB * CH)], semo).wait()
                pltpu.make_async_copy(
                    rowsa_v, outa_hbm.at[pl.ds(base, NB * CH)], semo).wait()

            descs = []
            for b in range(NB):
                descs.append(pltpu.async_copy(
                    src_hbm.at[idc_v.at[pl.ds(jo + b * CH, CH)]],
                    rowsc_v.at[pl.ds(b * CH, CH)], semg))
                descs.append(pltpu.async_copy(
                    src2_hbm.at[ida_v.at[pl.ds(jo + b * CH, CH)]],
                    rowsa_v.at[pl.ds(b * CH, CH)], semg))
            for d in descs:
                d.wait()
            pltpu.async_copy(rowsc_v, outc_hbm.at[pl.ds(base + jo, NB * CH)],
                             semo)
            pltpu.async_copy(rowsa_v, outa_hbm.at[pl.ds(base + jo, NB * CH)],
                             semo)
            return carry

        lax.fori_loop(0, NG, group, 0)
        if NG > 0:
            pltpu.make_async_copy(
                rowsc_v, outc_hbm.at[pl.ds(base, NB * CH)], semo).wait()
            pltpu.make_async_copy(
                rowsa_v, outa_hbm.at[pl.ds(base, NB * CH)], semo).wait()
        if tail:
            rows_tc, rows_ta = rest
            off = base + nfull * CH
            d1 = pltpu.async_copy(
                src_hbm.at[idc_v.at[pl.ds(nfull * CH, tail)]], rows_tc, semg)
            d2 = pltpu.async_copy(
                src2_hbm.at[ida_v.at[pl.ds(nfull * CH, tail)]], rows_ta,
                semg)
            d1.wait()
            d2.wait()
            pltpu.sync_copy(rows_tc, outc_hbm.at[pl.ds(off, tail)])
            pltpu.sync_copy(rows_ta, outa_hbm.at[pl.ds(off, tail)])

    return k(src1, src2, idx_c, idx_a)


def _sc_segsum_sorted(tmp, ids, zslab):
    T, E, W = N_TRIP, N_EDGES, 64
    RW = 16000
    NPASS = E // RW // 2
    CH = 128
    NCH = T // CH
    TPT = -(-NCH // 16)
    NB = 3
    NG = (TPT - 1) // NB
    R = 16016
    DUMP = 16000
    scratch = [pltpu.VMEM_SHARED((R, W), F32),
               pltpu.VMEM((TPT * CH,), I32),
               pltpu.VMEM((NB, CH), I32),
               pltpu.VMEM((NB * CH, W), F32),
               pltpu.SemaphoreType.DMA, pltpu.SemaphoreType.DMA,
               pltpu.SemaphoreType.DMA]

    @functools.partial(
        pl.kernel, out_type=jax.ShapeDtypeStruct((E, W), F32),
        mesh=plsc.VectorSubcoreMesh(**_MESH), scratch_types=scratch,
        compiler_params=_SC_PARAMS)
    def k(tmp_hbm, ids_hbm, z_hbm, out_hbm, slab, ids_v, idx2, vals,
          semp, semv, semsc):
        cid = lax.axis_index("c")
        sid = lax.axis_index("s")

        def src_row(t):
            return jnp.minimum(sid + 16 * t, NCH - 1) * CH

        def pre(t, carry):
            pltpu.async_copy(ids_hbm.at[pl.ds(src_row(t), CH)],
                             ids_v.at[pl.ds(t * CH, CH)], semp)

            @pl.when(t >= 8)
            def _():
                pltpu.make_async_copy(
                    ids_hbm.at[pl.ds(0, CH)], ids_v.at[pl.ds(0, CH)],
                    semp).wait()
            return carry

        lax.fori_loop(0, TPT, pre, 0)
        for _ in range(8):
            pltpu.make_async_copy(ids_hbm.at[pl.ds(0, CH)],
                                  ids_v.at[pl.ds(0, CH)], semp).wait()

        for p in range(NPASS):
            lo = (2 * p + cid) * RW
            zb = sid * (R // 16)
            pltpu.sync_copy(z_hbm.at[pl.ds(zb, R // 16)],
                            slab.at[pl.ds(zb, R // 16)])
            plsc.subcore_barrier()

            def active(t):
                minv = jnp.min(ids_v[pl.ds(t * CH, 16)])
                maxv = jnp.max(ids_v[pl.ds(t * CH + CH - 16, 16)])
                return ((maxv >= lo) & (minv < lo + RW)
                        & (sid + 16 * t < NCH))

            def fire(t, b):
                pltpu.async_copy(tmp_hbm.at[pl.ds(src_row(t), CH)],
                                 vals.at[pl.ds(b * CH, CH)], semv)

            def flush(t, b):
                pltpu.make_async_copy(tmp_hbm.at[pl.ds(src_row(t), CH)],
                                      vals.at[pl.ds(b * CH, CH)],
                                      semv).wait()
                row = idx2.at[b]
                for q in range(CH // 16):
                    iv = ids_v[pl.ds(t * CH + q * 16, 16)]
                    ok = (iv >= lo) & (iv < lo + RW)
                    row[pl.ds(q * 16, 16)] = jnp.where(ok, iv - lo, DUMP)
                pltpu.async_copy(vals.at[pl.ds(b * CH, CH)],
                                 slab.at[idx2.at[b]], semsc, add=True)

            def drain(b):
                pltpu.make_async_copy(vals.at[pl.ds(b * CH, CH)],
                                      slab.at[idx2.at[b]], semsc).wait()

            def group(g, carry):
                gp = jnp.maximum(g - 1, 0)
                for b in range(NB):
                    @pl.when((g > 0) & active(gp * NB + b))
                    def _(b=b):
                        drain(b)
                for b in range(NB):
                    t = g * NB + b

                    @pl.when(active(t))
                    def _(t=t, b=b):
                        fire(t, b)
                for b in range(NB):
                    t = g * NB + b

                    @pl.when(active(t))
                    def _(t=t, b=b):
                        flush(t, b)
                return carry

            lax.fori_loop(0, NG, group, 0)
            for b in range(NB):
                @pl.when(active((NG - 1) * NB + b))
                def _(b=b):
                    drain(b)
            t_ep = NG * NB
            for t in range(t_ep, TPT):
                @pl.when(active(t))
                def _(t=t):
                    fire(t, 0)
                    flush(t, 0)
                    drain(0)

            plsc.subcore_barrier()
            cb = sid * (RW // 16)
            pltpu.sync_copy(slab.at[pl.ds(cb, RW // 16)],
                            out_hbm.at[pl.ds(cb + lo, RW // 16)])
            plsc.subcore_barrier()

    return k(tmp, ids, zslab)


def _sc_scatter_atoms(xm, id_a, zslab):
    E, A, W = N_EDGES, N_ATOMS, 64
    per = E // 16
    CH = 128
    nfull = per // CH
    tail = per - nfull * CH
    NB = 6
    NG = nfull // NB
    R = 10016
    scratch = [pltpu.VMEM_SHARED((R, W), F32),
               pltpu.VMEM((NB, CH), I32), pltpu.VMEM((NB * CH, W), F32),
               pltpu.VMEM((tail,), I32), pltpu.VMEM((tail, W), F32),
               pltpu.SemaphoreType.DMA, pltpu.SemaphoreType.DMA,
               pltpu.SemaphoreType.DMA]

    @functools.partial(
        pl.kernel, out_type=jax.ShapeDtypeStruct((A, 2 * W), F32),
        mesh=plsc.VectorSubcoreMesh(**_MESH), scratch_types=scratch,
        compiler_params=_SC_PARAMS)
    def k(xm_hbm, ida_hbm, z_hbm, out_hbm, slab, idx2, vals, idx_t,
          val_t, semi, semv, semsc):
        cid = lax.axis_index("c")
        sid = lax.axis_index("s")
        col = cid * W
        zb = sid * (R // 16)
        pltpu.sync_copy(z_hbm.at[pl.ds(zb, R // 16)],
                        slab.at[pl.ds(zb, R // 16)])
        plsc.subcore_barrier()
        base = sid * per

        def group(g, carry):
            for b in range(NB):
                @pl.when(g > 0)
                def _(b=b):
                    pltpu.make_async_copy(
                        vals.at[pl.ds(b * CH, CH)],
                        slab.at[idx2.at[b]], semsc).wait()
            descs = []
            for b in range(NB):
                off = base + (g * NB + b) * CH
                descs.append(pltpu.async_copy(
                    ida_hbm.at[pl.ds(off, CH)], idx2.at[b], semi))
                descs.append(pltpu.async_copy(
                    xm_hbm.at[pl.ds(off, CH), pl.ds(col, W)],
                    vals.at[pl.ds(b * CH, CH)], semv))
            for d in descs:
                d.wait()
            for b in range(NB):
                pltpu.async_copy(vals.at[pl.ds(b * CH, CH)],
                                 slab.at[idx2.at[b]], semsc, add=True)
            return carry

        lax.fori_loop(0, NG, group, 0)
        for b in range(NB):
            pltpu.make_async_copy(vals.at[pl.ds(b * CH, CH)],
                                  slab.at[idx2.at[b]], semsc).wait()
        off = base + nfull * CH
        pltpu.sync_copy(ida_hbm.at[pl.ds(off, tail)], idx_t)
        pltpu.sync_copy(xm_hbm.at[pl.ds(off, tail), pl.ds(col, W)], val_t)
        pltpu.sync_copy(val_t, slab.at[idx_t], add=True)
        plsc.subcore_barrier()
        cb = sid * (A // 16)
        pltpu.sync_copy(slab.at[pl.ds(cb, A // 16)],
                        out_hbm.at[pl.ds(cb, A // 16), pl.ds(col, W)])

    return k(xm, id_a, zslab)


def kernel(h, m, rbf3, cbf3, Kidx3, id_swap, id3_expand_ba, id3_reduce_ca,
           rbf_h, id_c, id_a, W_dense_ca, W_dense_ba, W_mlp_rbf,
           W_bilinear, W_down, W_up_ca, W_up_ac, W_res_bef, W_res_aft,
           W_rbf_h, W_atom_dense1, W_atom_res, W_concat, W_res_m):
    Wb_t = jnp.transpose(W_bilinear, (1, 0, 2)).reshape(1024, 64)
    Wb_t = Wb_t.astype(jnp.bfloat16)
    i32 = lambda a: a.astype(I32)

    x_ba = _edge_pre(m, rbf3, W_dense_ba, W_mlp_rbf, W_down)
    x_ba_t = _sc_gather(x_ba, i32(id3_expand_ba))
    tmp = _bilinear(x_ba_t, cbf3, Wb_t)
    x = _sc_segsum_sorted(tmp, i32(id3_reduce_ca),
                          jnp.zeros((16016, 64), F32))
    x_sw = _sc_gather(x, i32(id_swap))
    m1, xm = _edge_mid(m, x, x_sw, rbf_h, W_dense_ca, W_up_ca, W_up_ac,
                       W_res_bef[0, 0], W_res_bef[0, 1],
                       W_res_aft[0, 0], W_res_aft[0, 1],
                       W_res_aft[1, 0], W_res_aft[1, 1], W_rbf_h)
    x2 = _sc_scatter_atoms(xm, i32(id_a), jnp.zeros((10016, 64), F32))
    h_new, g1, g2 = _atoms(x2, h, W_atom_dense1,
                           W_atom_res[0, 0], W_atom_res[0, 1],
                           W_atom_res[1, 0], W_atom_res[1, 1],
                           W_atom_res[2, 0], W_atom_res[2, 1],
                           W_concat[:128], W_concat[128:256])
    g1c, g2a = _sc_gather2(g1, i32(id_c), g2, i32(id_a))
    m_out = _edge_fin(g1c, g2a, m1, W_concat[256:], W_res_m[0, 0],
                      W_res_m[0, 1])
    return (h_new, m_out)


# speedup vs baseline: 1.1915x; 1.1915x over previous
import functools

import jax
import jax.numpy as jnp
from jax import lax
from jax.experimental import pallas as pl
from jax.experimental.pallas import tpu as pltpu
from jax.experimental.pallas import tpu_sc as plsc

F32 = jnp.float32
I32 = jnp.int32
INV_SQRT2 = 0.7071067811865475
ACT_SCALE = 1.0 / 0.6

N_ATOMS = 10000
N_EDGES = 160000
N_TRIP = 320000


def _act(x):
    return jax.nn.silu(x) * ACT_SCALE


def _res(x, w0, w1):
    y = _act(x @ w0[...])
    y = _act(y @ w1[...])
    return (x + y) * INV_SQRT2


def _edge_pre_kernel(m_ref, rbf3_ref, wba, wrbf, wdown, xba_ref):
    t = _act(m_ref[...] @ wba[...])
    t = t * (rbf3_ref[...] @ wrbf[...])
    xba_ref[...] = _act(t @ wdown[...])


def _edge_pre(m, rbf3, W_ba, W_rbf, W_down):
    E, BE = N_EDGES, 2000
    full = lambda s: pl.BlockSpec(s, lambda i: (0, 0))
    return pl.pallas_call(
        _edge_pre_kernel,
        grid=(E // BE,),
        in_specs=[pl.BlockSpec((BE, 128), lambda i: (i, 0)),
                  pl.BlockSpec((BE, 16), lambda i: (i, 0)),
                  full((128, 128)), full((16, 128)), full((128, 64))],
        out_specs=pl.BlockSpec((BE, 64), lambda i: (i, 0)),
        out_shape=jax.ShapeDtypeStruct((E, 64), F32),
    )(m, rbf3, W_ba, W_rbf, W_down)


def _bilinear_kernel(x_ref, c_ref, w_ref, out_ref, z_ref):
    x = x_ref[...].astype(jnp.bfloat16)
    c = c_ref[...].astype(jnp.bfloat16)
    z = jnp.concatenate([x * c[:, k:k + 1] for k in range(16)], axis=1)
    out_ref[...] = jnp.dot(z, w_ref[...], preferred_element_type=F32)


def _bilinear(x_ba_t, cbf3, Wb_t):
    T, BT = N_TRIP, 2000
    return pl.pallas_call(
        _bilinear_kernel,
        grid=(T // BT,),
        in_specs=[pl.BlockSpec((BT, 64), lambda i: (i, 0)),
                  pl.BlockSpec((BT, 16), lambda i: (i, 0)),
                  pl.BlockSpec((1024, 64), lambda i: (0, 0)),
                  ],
        out_specs=pl.BlockSpec((BT, 64), lambda i: (i, 0)),
        out_shape=jax.ShapeDtypeStruct((T, 64), F32),
        scratch_shapes=[pltpu.VMEM((BT, 1024), jnp.bfloat16)],
    )(x_ba_t, cbf3, Wb_t)


def _edge_mid_kernel(m_ref, x_ref, xsw_ref, rbfh_ref,
                     wca, wupca, wupac, bef0, bef1, a00, a01, a10, a11,
                     wrbfh, m1_ref, xm_ref):
    x_ca = _act(x_ref[...] @ wupca[...])
    x_ac = _act(xsw_ref[...] @ wupac[...])
    x3 = (x_ca + x_ac) * INV_SQRT2
    x4 = (_act(m_ref[...] @ wca[...]) + x3) * INV_SQRT2
    x4 = _res(x4, bef0, bef1)
    m1 = (m_ref[...] + x4) * INV_SQRT2
    m1 = _res(m1, a00, a01)
    m1 = _res(m1, a10, a11)
    m1_ref[...] = m1
    xm_ref[...] = m1 * (rbfh_ref[...] @ wrbfh[...])


def _edge_mid(m, x, x_sw, rbf_h, wca, wupca, wupac, bef0, bef1,
              a00, a01, a10, a11, wrbfh):
    E, BE = N_EDGES, 2000
    full = lambda s: pl.BlockSpec(s, lambda i: tuple(0 for _ in s))
    return pl.pallas_call(
        _edge_mid_kernel,
        grid=(E // BE,),
        in_specs=[pl.BlockSpec((BE, 128), lambda i: (i, 0)),
                  pl.BlockSpec((BE, 64), lambda i: (i, 0)),
                  pl.BlockSpec((BE, 64), lambda i: (i, 0)),
                  pl.BlockSpec((BE, 16), lambda i: (i, 0)),
                  full((128, 128)), full((64, 128)), full((64, 128)),
                  full((128, 128)), full((128, 128)), full((128, 128)),
                  full((128, 128)), full((128, 128)), full((128, 128)),
                  full((16, 128))],
        out_specs=[pl.BlockSpec((BE, 128), lambda i: (i, 0)),
                   pl.BlockSpec((BE, 128), lambda i: (i, 0))],
        out_shape=[jax.ShapeDtypeStruct((E, 128), F32),
                   jax.ShapeDtypeStruct((E, 128), F32)],
    )(m, x, x_sw, rbf_h, wca, wupca, wupac, bef0, bef1, a00, a01, a10,
      a11, wrbfh)


def _atom_kernel(x2_ref, h_ref, w1, r00, r01, r10, r11, r20, r21,
                 wc1, wc2, hout_ref, g1_ref, g2_ref):
    xh = _act(x2_ref[...] @ w1[...])
    xh = _res(xh, r00, r01)
    xh = _res(xh, r10, r11)
    xh = _res(xh, r20, r21)
    hn = (h_ref[...] + xh) * INV_SQRT2
    hout_ref[...] = hn
    g1_ref[...] = hn @ wc1[...]
    g2_ref[...] = hn @ wc2[...]


def _atoms(x2, h, w1, r00, r01, r10, r11, r20, r21, wc1, wc2):
    A, BA = N_ATOMS, 2000
    full = lambda s: pl.BlockSpec(s, lambda i: tuple(0 for _ in s))
    return pl.pallas_call(
        _atom_kernel,
        grid=(A // BA,),
        in_specs=[pl.BlockSpec((BA, 128), lambda i: (i, 0)),
                  pl.BlockSpec((BA, 128), lambda i: (i, 0)),
                  full((128, 128)), full((128, 128)), full((128, 128)),
                  full((128, 128)), full((128, 128)), full((128, 128)),
                  full((128, 128)), full((128, 128)), full((128, 128))],
        out_specs=[pl.BlockSpec((BA, 128), lambda i: (i, 0))] * 3,
        out_shape=[jax.ShapeDtypeStruct((A, 128), F32),
                   jax.ShapeDtypeStruct((A, 128), F32),
                   jax.ShapeDtypeStruct((A, 128), F32)],
    )(x2, h, w1, r00, r01, r10, r11, r20, r21, wc1, wc2)


def _edge_fin_kernel(g1_ref, g2_ref, m1_ref, wc3, rm0, rm1, out_ref):
    m1 = m1_ref[...]
    m2 = _act(g1_ref[...] + g2_ref[...] + m1 @ wc3[...])
    m2 = _res(m2, rm0, rm1)
    out_ref[...] = (m1 + m2) * INV_SQRT2


def _edge_fin(g1c, g2a, m1, wc3, rm0, rm1):
    E, BE = N_EDGES, 2000
    full = lambda s: pl.BlockSpec(s, lambda i: (0, 0))
    return pl.pallas_call(
        _edge_fin_kernel,
        grid=(E // BE,),
        in_specs=[pl.BlockSpec((BE, 128), lambda i: (i, 0)),
                  pl.BlockSpec((BE, 128), lambda i: (i, 0)),
                  pl.BlockSpec((BE, 128), lambda i: (i, 0)),
                  full((128, 128)), full((128, 128)), full((128, 128))],
        out_specs=pl.BlockSpec((BE, 128), lambda i: (i, 0)),
        out_shape=jax.ShapeDtypeStruct((E, 128), F32),
    )(g1c, g2a, m1, wc3, rm0, rm1)


_MESH = dict(core_axis_name="c", subcore_axis_name="s")
_SC_PARAMS = pltpu.CompilerParams(use_tc_tiling_on_sc=False,
                                  needs_layout_passes=False)


def _pick_nb(n):
    return next(nb for nb in (6, 4, 3, 2, 1) if n % nb == 0)


def _sc_gather(src, idx):
    N, W = src.shape
    dt = src.dtype
    B = idx.shape[0]
    per = B // 32
    CH = 128
    nfull = per // CH
    tail = per - nfull * CH
    NB = _pick_nb(nfull)
    NG = nfull // NB
    scratch = [pltpu.VMEM((per,), I32), pltpu.VMEM((NB * CH, W), dt),
               pltpu.SemaphoreType.DMA, pltpu.SemaphoreType.DMA]
    if tail:
        scratch += [pltpu.VMEM((tail, W), dt)]

    @functools.partial(
        pl.kernel, out_type=jax.ShapeDtypeStruct((B, W), dt),
        mesh=plsc.VectorSubcoreMesh(**_MESH), scratch_types=scratch,
        compiler_params=_SC_PARAMS)
    def k(src_hbm, idx_hbm, out_hbm, ids_v, rows_v, semg, semo, *rest):
        wid = lax.axis_index("s") * 2 + lax.axis_index("c")
        base = wid * per
        pltpu.sync_copy(idx_hbm.at[pl.ds(base, per)], ids_v)

        def group(g, carry):
            jo = g * NB * CH

            @pl.when(g > 0)
            def _():
                pltpu.make_async_copy(
                    rows_v, out_hbm.at[pl.ds(base, NB * CH)], semo).wait()

            descs = []
            for b in range(NB):
                descs.append(pltpu.async_copy(
                    src_hbm.at[ids_v.at[pl.ds(jo + b * CH, CH)]],
                    rows_v.at[pl.ds(b * CH, CH)], semg))
            for d in descs:
                d.wait()
            pltpu.async_copy(rows_v, out_hbm.at[pl.ds(base + jo, NB * CH)],
                             semo)
            return carry

        lax.fori_loop(0, NG, group, 0)
        if NG > 0:
            pltpu.make_async_copy(
                rows_v, out_hbm.at[pl.ds(base, NB * CH)], semo).wait()
        if tail:
            (rows_t,) = rest
            off = base + nfull * CH
            pltpu.async_copy(
                src_hbm.at[ids_v.at[pl.ds(nfull * CH, tail)]],
                rows_t, semg).wait()
            pltpu.sync_copy(rows_t, out_hbm.at[pl.ds(off, tail)])

    return k(src, idx)


def _sc_gather2(src1, idx_c, src2, idx_a):
    N, W = src1.shape
    dt = src1.dtype
    B = idx_c.shape[0]
    per = B // 32
    CH = 128
    nfull = per // CH
    tail = per - nfull * CH
    NB = _pick_nb(nfull)
    NG = nfull // NB
    out_t = [jax.ShapeDtypeStruct((B, W), dt)] * 2
    scratch = [pltpu.VMEM((per,), I32), pltpu.VMEM((per,), I32),
               pltpu.VMEM((NB * CH, W), dt), pltpu.VMEM((NB * CH, W), dt),
               pltpu.SemaphoreType.DMA, pltpu.SemaphoreType.DMA]
    if tail:
        scratch += [pltpu.VMEM((tail, W), dt), pltpu.VMEM((tail, W), dt)]

    @functools.partial(
        pl.kernel, out_type=out_t,
        mesh=plsc.VectorSubcoreMesh(**_MESH), scratch_types=scratch,
        compiler_params=_SC_PARAMS)
    def k(src_hbm, src2_hbm, idxc_hbm, idxa_hbm, outc_hbm, outa_hbm,
          idc_v, ida_v, rowsc_v, rowsa_v, semg, semo, *rest):
        wid = lax.axis_index("s") * 2 + lax.axis_index("c")
        base = wid * per
        pltpu.sync_copy(idxc_hbm.at[pl.ds(base, per)], idc_v)
        pltpu.sync_copy(idxa_hbm.at[pl.ds(base, per)], ida_v)

        def group(g, carry):
            jo = g * NB * CH

            @pl.when(g > 0)
            def _():
                pltpu.make_async_copy(
                    rowsc_v, outc_hbm.at[pl.ds(base, NB * CH)], semo).wait()
                pltpu.make_async_copy(
                    rowsa_v, outa_hbm.at[pl.ds(base, NB * CH)], semo).wait()

            descs = []
            for b in range(NB):
                descs.append(pltpu.async_copy(
                    src_hbm.at[idc_v.at[pl.ds(jo + b * CH, CH)]],
                    rowsc_v.at[pl.ds(b * CH, CH)], semg))
                descs.append(pltpu.async_copy(
                    src2_hbm.at[ida_v.at[pl.ds(jo + b * CH, CH)]],
                    rowsa_v.at[pl.ds(b * CH, CH)], semg))
            for d in descs:
                d.wait()
            pltpu.async_copy(rowsc_v, outc_hbm.at[pl.ds(base + jo, NB * CH)],
                             semo)
            pltpu.async_copy(rowsa_v, outa_hbm.at[pl.ds(base + jo, NB * CH)],
                             semo)
            return carry

        lax.fori_loop(0, NG, group, 0)
        if NG > 0:
            pltpu.make_async_copy(
                rowsc_v, outc_hbm.at[pl.ds(base, NB * CH)], semo).wait()
            pltpu.make_async_copy(
                rowsa_v, outa_hbm.at[pl.ds(base, NB * CH)], semo).wait()
        if tail:
            rows_tc, rows_ta = rest
            off = base + nfull * CH
            d1 = pltpu.async_copy(
                src_hbm.at[idc_v.at[pl.ds(nfull * CH, tail)]], rows_tc, semg)
            d2 = pltpu.async_copy(
                src2_hbm.at[ida_v.at[pl.ds(nfull * CH, tail)]], rows_ta,
                semg)
            d1.wait()
            d2.wait()
            pltpu.sync_copy(rows_tc, outc_hbm.at[pl.ds(off, tail)])
            pltpu.sync_copy(rows_ta, outa_hbm.at[pl.ds(off, tail)])

    return k(src1, src2, idx_c, idx_a)


def _sc_segsum_sorted(tmp, ids, zslab):
    T, E, W = N_TRIP, N_EDGES, 64
    RW = 16000
    NPASS = E // RW // 2
    CH = 128
    NCH = T // CH
    TPT = -(-NCH // 16)
    NB = 3
    NG = (TPT - 1) // NB
    R = 16016
    DUMP = 16000
    scratch = [pltpu.VMEM_SHARED((R, W), F32),
               pltpu.VMEM((TPT * CH,), I32),
               pltpu.VMEM((NB, CH), I32),
               pltpu.VMEM((NB * CH, W), F32),
               pltpu.SemaphoreType.DMA, pltpu.SemaphoreType.DMA,
               pltpu.SemaphoreType.DMA]

    @functools.partial(
        pl.kernel, out_type=jax.ShapeDtypeStruct((E, W), F32),
        mesh=plsc.VectorSubcoreMesh(**_MESH), scratch_types=scratch,
        compiler_params=_SC_PARAMS)
    def k(tmp_hbm, ids_hbm, z_hbm, out_hbm, slab, ids_v, idx2, vals,
          semp, semv, semsc):
        cid = lax.axis_index("c")
        sid = lax.axis_index("s")

        def src_row(t):
            return jnp.minimum(sid + 16 * t, NCH - 1) * CH

        def pre(t, carry):
            pltpu.async_copy(ids_hbm.at[pl.ds(src_row(t), CH)],
                             ids_v.at[pl.ds(t * CH, CH)], semp)

            @pl.when(t >= 8)
            def _():
                pltpu.make_async_copy(
                    ids_hbm.at[pl.ds(0, CH)], ids_v.at[pl.ds(0, CH)],
                    semp).wait()
            return carry

        lax.fori_loop(0, TPT, pre, 0)
        for _ in range(8):
            pltpu.make_async_copy(ids_hbm.at[pl.ds(0, CH)],
                                  ids_v.at[pl.ds(0, CH)], semp).wait()

        for p in range(NPASS):
            lo = (2 * p + cid) * RW
            zb = sid * (R // 16)
            pltpu.sync_copy(z_hbm.at[pl.ds(zb, R // 16)],
                            slab.at[pl.ds(zb, R // 16)])
            plsc.subcore_barrier()

            def active(t):
                minv = jnp.min(ids_v[pl.ds(t * CH, 16)])
                maxv = jnp.max(ids_v[pl.ds(t * CH + CH - 16, 16)])
                return ((maxv >= lo) & (minv < lo + RW)
                        & (sid + 16 * t < NCH))

            def fire(t, b):
                pltpu.async_copy(tmp_hbm.at[pl.ds(src_row(t), CH)],
                                 vals.at[pl.ds(b * CH, CH)], semv)

            def flush(t, b):
                pltpu.make_async_copy(tmp_hbm.at[pl.ds(src_row(t), CH)],
                                      vals.at[pl.ds(b * CH, CH)],
                                      semv).wait()
                row = idx2.at[b]
                for q in range(CH // 16):
                    iv = ids_v[pl.ds(t * CH + q * 16, 16)]
                    ok = (iv >= lo) & (iv < lo + RW)
                    row[pl.ds(q * 16, 16)] = jnp.where(ok, iv - lo, DUMP)
                pltpu.async_copy(vals.at[pl.ds(b * CH, CH)],
                                 slab.at[idx2.at[b]], semsc, add=True)

            def drain(b):
                pltpu.make_async_copy(vals.at[pl.ds(b * CH, CH)],
                                      slab.at[idx2.at[b]], semsc).wait()

            def group(g, carry):
                gp = jnp.maximum(g - 1, 0)
                for b in range(NB):
                    @pl.when((g > 0) & active(gp * NB + b))
                    def _(b=b):
                        drain(b)
                for b in range(NB):
                    t = g * NB + b

                    @pl.when(active(t))
                    def _(t=t, b=b):
                        fire(t, b)
                for b in range(NB):
                    t = g * NB + b

                    @pl.when(active(t))
                    def _(t=t, b=b):
                        flush(t, b)
                return carry

            lax.fori_loop(0, NG, group, 0)
            for b in range(NB):
                @pl.when(active((NG - 1) * NB + b))
                def _(b=b):
                    drain(b)
            t_ep = NG * NB
            for t in range(t_ep, TPT):
                @pl.when(active(t))
                def _(t=t):
                    fire(t, 0)
                    flush(t, 0)
                    drain(0)

            plsc.subcore_barrier()
            cb = sid * (RW // 16)
            pltpu.sync_copy(slab.at[pl.ds(cb, RW // 16)],
                            out_hbm.at[pl.ds(cb + lo, RW // 16)])
            plsc.subcore_barrier()

    return k(tmp, ids, zslab)


def _sc_scatter_atoms(xm, id_a, zslab):
    E, A, W = N_EDGES, N_ATOMS, 64
    per = E // 16
    CH = 128
    nfull = per // CH
    tail = per - nfull * CH
    NB = 6
    NG = nfull // NB
    R = 10016
    scratch = [pltpu.VMEM_SHARED((R, W), F32),
               pltpu.VMEM((NB, CH), I32), pltpu.VMEM((NB * CH, W), F32),
               pltpu.VMEM((tail,), I32), pltpu.VMEM((tail, W), F32),
               pltpu.SemaphoreType.DMA, pltpu.SemaphoreType.DMA,
               pltpu.SemaphoreType.DMA]

    @functools.partial(
        pl.kernel, out_type=jax.ShapeDtypeStruct((A, 2 * W), F32),
        mesh=plsc.VectorSubcoreMesh(**_MESH), scratch_types=scratch,
        compiler_params=_SC_PARAMS)
    def k(xm_hbm, ida_hbm, z_hbm, out_hbm, slab, idx2, vals, idx_t,
          val_t, semi, semv, semsc):
        cid = lax.axis_index("c")
        sid = lax.axis_index("s")
        col = cid * W
        zb = sid * (R // 16)
        pltpu.sync_copy(z_hbm.at[pl.ds(zb, R // 16)],
                        slab.at[pl.ds(zb, R // 16)])
        plsc.subcore_barrier()
        base = sid * per

        def group(g, carry):
            for b in range(NB):
                @pl.when(g > 0)
                def _(b=b):
                    pltpu.make_async_copy(
                        vals.at[pl.ds(b * CH, CH)],
                        slab.at[idx2.at[b]], semsc).wait()
            descs = []
            for b in range(NB):
                off = base + (g * NB + b) * CH
                descs.append(pltpu.async_copy(
                    ida_hbm.at[pl.ds(off, CH)], idx2.at[b], semi))
                descs.append(pltpu.async_copy(
                    xm_hbm.at[pl.ds(off, CH), pl.ds(col, W)],
                    vals.at[pl.ds(b * CH, CH)], semv))
            for d in descs:
                d.wait()
            for b in range(NB):
                pltpu.async_copy(vals.at[pl.ds(b * CH, CH)],
                                 slab.at[idx2.at[b]], semsc, add=True)
            return carry

        lax.fori_loop(0, NG, group, 0)
        for b in range(NB):
            pltpu.make_async_copy(vals.at[pl.ds(b * CH, CH)],
                                  slab.at[idx2.at[b]], semsc).wait()
        off = base + nfull * CH
        pltpu.sync_copy(ida_hbm.at[pl.ds(off, tail)], idx_t)
        pltpu.sync_copy(xm_hbm.at[pl.ds(off, tail), pl.ds(col, W)], val_t)
        pltpu.sync_copy(val_t, slab.at[idx_t], add=True)
        plsc.subcore_barrier()
        cb = sid * (A // 16)
        pltpu.sync_copy(slab.at[pl.ds(cb, A // 16)],
                        out_hbm.at[pl.ds(cb, A // 16), pl.ds(col, W)])

    return k(xm, id_a, zslab)


def kernel(h, m, rbf3, cbf3, Kidx3, id_swap, id3_expand_ba, id3_reduce_ca,
           rbf_h, id_c, id_a, W_dense_ca, W_dense_ba, W_mlp_rbf,
           W_bilinear, W_down, W_up_ca, W_up_ac, W_res_bef, W_res_aft,
           W_rbf_h, W_atom_dense1, W_atom_res, W_concat, W_res_m):
    Wb_t = jnp.transpose(W_bilinear, (1, 0, 2)).reshape(1024, 64)
    Wb_t = Wb_t.astype(jnp.bfloat16)
    i32 = lambda a: a.astype(I32)

    x_ba = _edge_pre(m, rbf3, W_dense_ba, W_mlp_rbf, W_down)
    x_ba_t = _sc_gather(x_ba, i32(id3_expand_ba))
    tmp = _bilinear(x_ba_t, cbf3, Wb_t)
    x = _sc_segsum_sorted(tmp, i32(id3_reduce_ca),
                          jnp.zeros((16016, 64), F32))
    x_sw = _sc_gather(x, i32(id_swap))
    m1, xm = _edge_mid(m, x, x_sw, rbf_h, W_dense_ca, W_up_ca, W_up_ac,
                       W_res_bef[0, 0], W_res_bef[0, 1],
                       W_res_aft[0, 0], W_res_aft[0, 1],
                       W_res_aft[1, 0], W_res_aft[1, 1], W_rbf_h)
    x2 = _sc_scatter_atoms(xm, i32(id_a), jnp.zeros((10016, 64), F32))
    h_new, g1, g2 = _atoms(x2, h, W_atom_dense1,
                           W_atom_res[0, 0], W_atom_res[0, 1],
                           W_atom_res[1, 0], W_atom_res[1, 1],
                           W_atom_res[2, 0], W_atom_res[2, 1],
                           W_concat[:128], W_concat[128:256])
    g1c, g2a = _sc_gather2(g1, i32(id_c), g2, i32(id_a))
    m_out = _edge_fin(g1c, g2a, m1, W_concat[256:], W_res_m[0, 0],
                      W_res_m[0, 1])
    return (h_new, m_out)


# speedup vs baseline: 1.2151x; 1.0198x over previous
import functools

import jax
import jax.numpy as jnp
from jax import lax
from jax.experimental import pallas as pl
from jax.experimental.pallas import tpu as pltpu
from jax.experimental.pallas import tpu_sc as plsc

F32 = jnp.float32
I32 = jnp.int32
INV_SQRT2 = 0.7071067811865475
ACT_SCALE = 1.0 / 0.6

N_ATOMS = 10000
N_EDGES = 160000
N_TRIP = 320000


def _act(x):
    return jax.nn.silu(x) * ACT_SCALE


def _res(x, w0, w1):
    y = _act(x @ w0[...])
    y = _act(y @ w1[...])
    return (x + y) * INV_SQRT2


def _edge_pre_kernel(m_ref, rbf3_ref, wba, wrbf, wdown, xba_ref):
    t = _act(m_ref[...] @ wba[...])
    t = t * (rbf3_ref[...] @ wrbf[...])
    xba_ref[...] = _act(t @ wdown[...])


def _edge_pre(m, rbf3, W_ba, W_rbf, W_down):
    E, BE = N_EDGES, 2000
    full = lambda s: pl.BlockSpec(s, lambda i: (0, 0))
    return pl.pallas_call(
        _edge_pre_kernel,
        grid=(E // BE,),
        in_specs=[pl.BlockSpec((BE, 128), lambda i: (i, 0)),
                  pl.BlockSpec((BE, 16), lambda i: (i, 0)),
                  full((128, 128)), full((16, 128)), full((128, 64))],
        out_specs=pl.BlockSpec((BE, 64), lambda i: (i, 0)),
        out_shape=jax.ShapeDtypeStruct((E, 64), F32),
    )(m, rbf3, W_ba, W_rbf, W_down)


def _bilinear_kernel(x_ref, c_ref, w_ref, out_ref, z_ref):
    x = x_ref[...].astype(jnp.bfloat16)
    c = c_ref[...].astype(jnp.bfloat16)
    z = jnp.concatenate([x * c[:, k:k + 1] for k in range(16)], axis=1)
    out_ref[...] = jnp.dot(z, w_ref[...], preferred_element_type=F32)


def _bilinear(x_ba_t, cbf3, Wb_t):
    T, BT = N_TRIP, 4000
    return pl.pallas_call(
        _bilinear_kernel,
        grid=(T // BT,),
        in_specs=[pl.BlockSpec((BT, 64), lambda i: (i, 0)),
                  pl.BlockSpec((BT, 16), lambda i: (i, 0)),
                  pl.BlockSpec((1024, 64), lambda i: (0, 0)),
                  ],
        out_specs=pl.BlockSpec((BT, 64), lambda i: (i, 0)),
        out_shape=jax.ShapeDtypeStruct((T, 64), F32),
        scratch_shapes=[pltpu.VMEM((BT, 1024), jnp.bfloat16)],
    )(x_ba_t, cbf3, Wb_t)


def _edge_mid_kernel(m_ref, x_ref, xsw_ref, rbfh_ref,
                     wca, wupca, wupac, bef0, bef1, a00, a01, a10, a11,
                     wrbfh, m1_ref, xm_ref):
    x_ca = _act(x_ref[...] @ wupca[...])
    x_ac = _act(xsw_ref[...] @ wupac[...])
    x3 = (x_ca + x_ac) * INV_SQRT2
    x4 = (_act(m_ref[...] @ wca[...]) + x3) * INV_SQRT2
    x4 = _res(x4, bef0, bef1)
    m1 = (m_ref[...] + x4) * INV_SQRT2
    m1 = _res(m1, a00, a01)
    m1 = _res(m1, a10, a11)
    m1_ref[...] = m1
    xm_ref[...] = m1 * (rbfh_ref[...] @ wrbfh[...])


def _edge_mid(m, x, x_sw, rbf_h, wca, wupca, wupac, bef0, bef1,
              a00, a01, a10, a11, wrbfh):
    E, BE = N_EDGES, 2000
    full = lambda s: pl.BlockSpec(s, lambda i: tuple(0 for _ in s))
    return pl.pallas_call(
        _edge_mid_kernel,
        grid=(E // BE,),
        in_specs=[pl.BlockSpec((BE, 128), lambda i: (i, 0)),
                  pl.BlockSpec((BE, 64), lambda i: (i, 0)),
                  pl.BlockSpec((BE, 64), lambda i: (i, 0)),
                  pl.BlockSpec((BE, 16), lambda i: (i, 0)),
                  full((128, 128)), full((64, 128)), full((64, 128)),
                  full((128, 128)), full((128, 128)), full((128, 128)),
                  full((128, 128)), full((128, 128)), full((128, 128)),
                  full((16, 128))],
        out_specs=[pl.BlockSpec((BE, 128), lambda i: (i, 0)),
                   pl.BlockSpec((BE, 128), lambda i: (i, 0))],
        out_shape=[jax.ShapeDtypeStruct((E, 128), F32),
                   jax.ShapeDtypeStruct((E, 128), F32)],
    )(m, x, x_sw, rbf_h, wca, wupca, wupac, bef0, bef1, a00, a01, a10,
      a11, wrbfh)


def _atom_kernel(x2_ref, h_ref, w1, r00, r01, r10, r11, r20, r21,
                 wc1, wc2, hout_ref, g1_ref, g2_ref):
    xh = _act(x2_ref[...] @ w1[...])
    xh = _res(xh, r00, r01)
    xh = _res(xh, r10, r11)
    xh = _res(xh, r20, r21)
    hn = (h_ref[...] + xh) * INV_SQRT2
    hout_ref[...] = hn
    g1_ref[...] = hn @ wc1[...]
    g2_ref[...] = hn @ wc2[...]


def _atoms(x2, h, w1, r00, r01, r10, r11, r20, r21, wc1, wc2):
    A, BA = N_ATOMS, 2000
    full = lambda s: pl.BlockSpec(s, lambda i: tuple(0 for _ in s))
    return pl.pallas_call(
        _atom_kernel,
        grid=(A // BA,),
        in_specs=[pl.BlockSpec((BA, 128), lambda i: (i, 0)),
                  pl.BlockSpec((BA, 128), lambda i: (i, 0)),
                  full((128, 128)), full((128, 128)), full((128, 128)),
                  full((128, 128)), full((128, 128)), full((128, 128)),
                  full((128, 128)), full((128, 128)), full((128, 128))],
        out_specs=[pl.BlockSpec((BA, 128), lambda i: (i, 0))] * 3,
        out_shape=[jax.ShapeDtypeStruct((A, 128), F32),
                   jax.ShapeDtypeStruct((A, 128), F32),
                   jax.ShapeDtypeStruct((A, 128), F32)],
    )(x2, h, w1, r00, r01, r10, r11, r20, r21, wc1, wc2)


def _edge_fin_kernel(g1_ref, g2_ref, m1_ref, wc3, rm0, rm1, out_ref):
    m1 = m1_ref[...]
    m2 = _act(g1_ref[...] + g2_ref[...] + m1 @ wc3[...])
    m2 = _res(m2, rm0, rm1)
    out_ref[...] = (m1 + m2) * INV_SQRT2


def _edge_fin(g1c, g2a, m1, wc3, rm0, rm1):
    E, BE = N_EDGES, 2000
    full = lambda s: pl.BlockSpec(s, lambda i: (0, 0))
    return pl.pallas_call(
        _edge_fin_kernel,
        grid=(E // BE,),
        in_specs=[pl.BlockSpec((BE, 128), lambda i: (i, 0)),
                  pl.BlockSpec((BE, 128), lambda i: (i, 0)),
                  pl.BlockSpec((BE, 128), lambda i: (i, 0)),
                  full((128, 128)), full((128, 128)), full((128, 128))],
        out_specs=pl.BlockSpec((BE, 128), lambda i: (i, 0)),
        out_shape=jax.ShapeDtypeStruct((E, 128), F32),
    )(g1c, g2a, m1, wc3, rm0, rm1)


_MESH = dict(core_axis_name="c", subcore_axis_name="s")
_SC_PARAMS = pltpu.CompilerParams(use_tc_tiling_on_sc=False,
                                  needs_layout_passes=False)


def _pick_nb(n):
    return next(nb for nb in (6, 4, 3, 2, 1) if n % nb == 0)


def _sc_gather(src, idx):
    N, W = src.shape
    dt = src.dtype
    B = idx.shape[0]
    per = B // 32
    CH = 128
    nfull = per // CH
    tail = per - nfull * CH
    NB = _pick_nb(nfull)
    NG = nfull // NB
    scratch = [pltpu.VMEM((per,), I32), pltpu.VMEM((NB * CH, W), dt),
               pltpu.SemaphoreType.DMA, pltpu.SemaphoreType.DMA]
    if tail:
        scratch += [pltpu.VMEM((tail, W), dt)]

    @functools.partial(
        pl.kernel, out_type=jax.ShapeDtypeStruct((B, W), dt),
        mesh=plsc.VectorSubcoreMesh(**_MESH), scratch_types=scratch,
        compiler_params=_SC_PARAMS)
    def k(src_hbm, idx_hbm, out_hbm, ids_v, rows_v, semg, semo, *rest):
        wid = lax.axis_index("s") * 2 + lax.axis_index("c")
        base = wid * per
        pltpu.sync_copy(idx_hbm.at[pl.ds(base, per)], ids_v)

        def group(g, carry):
            jo = g * NB * CH

            @pl.when(g > 0)
            def _():
                pltpu.make_async_copy(
                    rows_v, out_hbm.at[pl.ds(base, NB * CH)], semo).wait()

            descs = []
            for b in range(NB):
                descs.append(pltpu.async_copy(
                    src_hbm.at[ids_v.at[pl.ds(jo + b * CH, CH)]],
                    rows_v.at[pl.ds(b * CH, CH)], semg))
            for d in descs:
                d.wait()
            pltpu.async_copy(rows_v, out_hbm.at[pl.ds(base + jo, NB * CH)],
                             semo)
            return carry

        lax.fori_loop(0, NG, group, 0)
        if NG > 0:
            pltpu.make_async_copy(
                rows_v, out_hbm.at[pl.ds(base, NB * CH)], semo).wait()
        if tail:
            (rows_t,) = rest
            off = base + nfull * CH
            pltpu.async_copy(
                src_hbm.at[ids_v.at[pl.ds(nfull * CH, tail)]],
                rows_t, semg).wait()
            pltpu.sync_copy(rows_t, out_hbm.at[pl.ds(off, tail)])

    return k(src, idx)


def _sc_gather2(src1, idx_c, src2, idx_a):
    N, W = src1.shape
    dt = src1.dtype
    B = idx_c.shape[0]
    per = B // 32
    CH = 128
    nfull = per // CH
    tail = per - nfull * CH
    NB = _pick_nb(nfull)
    NG = nfull // NB
    out_t = [jax.ShapeDtypeStruct((B, W), dt)] * 2
    scratch = [pltpu.VMEM((per,), I32), pltpu.VMEM((per,), I32),
               pltpu.VMEM((NB * CH, W), dt), pltpu.VMEM((NB * CH, W), dt),
               pltpu.SemaphoreType.DMA, pltpu.SemaphoreType.DMA]
    if tail:
        scratch += [pltpu.VMEM((tail, W), dt), pltpu.VMEM((tail, W), dt)]

    @functools.partial(
        pl.kernel, out_type=out_t,
        mesh=plsc.VectorSubcoreMesh(**_MESH), scratch_types=scratch,
        compiler_params=_SC_PARAMS)
    def k(src_hbm, src2_hbm, idxc_hbm, idxa_hbm, outc_hbm, outa_hbm,
          idc_v, ida_v, rowsc_v, rowsa_v, semg, semo, *rest):
        wid = lax.axis_index("s") * 2 + lax.axis_index("c")
        base = wid * per
        pltpu.sync_copy(idxc_hbm.at[pl.ds(base, per)], idc_v)
        pltpu.sync_copy(idxa_hbm.at[pl.ds(base, per)], ida_v)

        def group(g, carry):
            jo = g * NB * CH

            @pl.when(g > 0)
            def _():
                pltpu.make_async_copy(
                    rowsc_v, outc_hbm.at[pl.ds(base, NB * CH)], semo).wait()
                pltpu.make_async_copy(
                    rowsa_v, outa_hbm.at[pl.ds(base, NB * CH)], semo).wait()

            descs = []
            for b in range(NB):
                descs.append(pltpu.async_copy(
                    src_hbm.at[idc_v.at[pl.ds(jo + b * CH, CH)]],
                    rowsc_v.at[pl.ds(b * CH, CH)], semg))
                descs.append(pltpu.async_copy(
                    src2_hbm.at[ida_v.at[pl.ds(jo + b * CH, CH)]],
                    rowsa_v.at[pl.ds(b * CH, CH)], semg))
            for d in descs:
                d.wait()
            pltpu.async_copy(rowsc_v, outc_hbm.at[pl.ds(base + jo, NB * CH)],
                             semo)
            pltpu.async_copy(rowsa_v, outa_hbm.at[pl.ds(base + jo, NB * CH)],
                             semo)
            return carry

        lax.fori_loop(0, NG, group, 0)
        if NG > 0:
            pltpu.make_async_copy(
                rowsc_v, outc_hbm.at[pl.ds(base, NB * CH)], semo).wait()
            pltpu.make_async_copy(
                rowsa_v, outa_hbm.at[pl.ds(base, NB * CH)], semo).wait()
        if tail:
            rows_tc, rows_ta = rest
            off = base + nfull * CH
            d1 = pltpu.async_copy(
                src_hbm.at[idc_v.at[pl.ds(nfull * CH, tail)]], rows_tc, semg)
            d2 = pltpu.async_copy(
                src2_hbm.at[ida_v.at[pl.ds(nfull * CH, tail)]], rows_ta,
                semg)
            d1.wait()
            d2.wait()
            pltpu.sync_copy(rows_tc, outc_hbm.at[pl.ds(off, tail)])
            pltpu.sync_copy(rows_ta, outa_hbm.at[pl.ds(off, tail)])

    return k(src1, src2, idx_c, idx_a)


def _sc_segsum_sorted(tmp, ids, zslab):
    T, E, W = N_TRIP, N_EDGES, 64
    RW = 16000
    NPASS = E // RW // 2
    CH = 128
    NCH = T // CH
    TPT = -(-NCH // 16)
    NB = 3
    NG = (TPT - 1) // NB
    R = 16016
    DUMP = 16000
    scratch = [pltpu.VMEM_SHARED((R, W), F32),
               pltpu.VMEM((TPT * CH,), I32),
               pltpu.VMEM((NB, CH), I32),
               pltpu.VMEM((NB * CH, W), F32),
               pltpu.SemaphoreType.DMA, pltpu.SemaphoreType.DMA,
               pltpu.SemaphoreType.DMA]

    @functools.partial(
        pl.kernel, out_type=jax.ShapeDtypeStruct((E, W), F32),
        mesh=plsc.VectorSubcoreMesh(**_MESH), scratch_types=scratch,
        compiler_params=_SC_PARAMS)
    def k(tmp_hbm, ids_hbm, z_hbm, out_hbm, slab, ids_v, idx2, vals,
          semp, semv, semsc):
        cid = lax.axis_index("c")
        sid = lax.axis_index("s")

        def src_row(t):
            return jnp.minimum(sid + 16 * t, NCH - 1) * CH

        def pre(t, carry):
            pltpu.async_copy(ids_hbm.at[pl.ds(src_row(t), CH)],
                             ids_v.at[pl.ds(t * CH, CH)], semp)

            @pl.when(t >= 8)
            def _():
                pltpu.make_async_copy(
                    ids_hbm.at[pl.ds(0, CH)], ids_v.at[pl.ds(0, CH)],
                    semp).wait()
            return carry

        lax.fori_loop(0, TPT, pre, 0)
        for _ in range(8):
            pltpu.make_async_copy(ids_hbm.at[pl.ds(0, CH)],
                                  ids_v.at[pl.ds(0, CH)], semp).wait()

        for p in range(NPASS):
            lo = (2 * p + cid) * RW
            zb = sid * (R // 16)
            pltpu.sync_copy(z_hbm.at[pl.ds(zb, R // 16)],
                            slab.at[pl.ds(zb, R // 16)])
            plsc.subcore_barrier()

            def active(t):
                minv = jnp.min(ids_v[pl.ds(t * CH, 16)])
                maxv = jnp.max(ids_v[pl.ds(t * CH + CH - 16, 16)])
                return ((maxv >= lo) & (minv < lo + RW)
                        & (sid + 16 * t < NCH))

            def fire(t, b):
                pltpu.async_copy(tmp_hbm.at[pl.ds(src_row(t), CH)],
                                 vals.at[pl.ds(b * CH, CH)], semv)

            def flush(t, b):
                pltpu.make_async_copy(tmp_hbm.at[pl.ds(src_row(t), CH)],
                                      vals.at[pl.ds(b * CH, CH)],
                                      semv).wait()
                row = idx2.at[b]
                for q in range(CH // 16):
                    iv = ids_v[pl.ds(t * CH + q * 16, 16)]
                    ok = (iv >= lo) & (iv < lo + RW)
                    row[pl.ds(q * 16, 16)] = jnp.where(ok, iv - lo, DUMP)
                pltpu.async_copy(vals.at[pl.ds(b * CH, CH)],
                                 slab.at[idx2.at[b]], semsc, add=True)

            def drain(b):
                pltpu.make_async_copy(vals.at[pl.ds(b * CH, CH)],
                                      slab.at[idx2.at[b]], semsc).wait()

            def group(g, carry):
                gp = jnp.maximum(g - 1, 0)
                for b in range(NB):
                    @pl.when((g > 0) & active(gp * NB + b))
                    def _(b=b):
                        drain(b)
                for b in range(NB):
                    t = g * NB + b

                    @pl.when(active(t))
                    def _(t=t, b=b):
                        fire(t, b)
                for b in range(NB):
                    t = g * NB + b

                    @pl.when(active(t))
                    def _(t=t, b=b):
                        flush(t, b)
                return carry

            lax.fori_loop(0, NG, group, 0)
            for b in range(NB):
                @pl.when(active((NG - 1) * NB + b))
                def _(b=b):
                    drain(b)
            t_ep = NG * NB
            for t in range(t_ep, TPT):
                @pl.when(active(t))
                def _(t=t):
                    fire(t, 0)
                    flush(t, 0)
                    drain(0)

            plsc.subcore_barrier()
            cb = sid * (RW // 16)
            pltpu.sync_copy(slab.at[pl.ds(cb, RW // 16)],
                            out_hbm.at[pl.ds(cb + lo, RW // 16)])
            plsc.subcore_barrier()

    return k(tmp, ids, zslab)


def _sc_scatter_atoms(xm, id_a, zslab):
    E, A, W = N_EDGES, N_ATOMS, 64
    per = E // 16
    CH = 128
    nfull = per // CH
    tail = per - nfull * CH
    NB = 6
    NG = nfull // NB
    R = 10016
    scratch = [pltpu.VMEM_SHARED((R, W), F32),
               pltpu.VMEM((NB, CH), I32), pltpu.VMEM((NB * CH, W), F32),
               pltpu.VMEM((tail,), I32), pltpu.VMEM((tail, W), F32),
               pltpu.SemaphoreType.DMA, pltpu.SemaphoreType.DMA,
               pltpu.SemaphoreType.DMA]

    @functools.partial(
        pl.kernel, out_type=jax.ShapeDtypeStruct((A, 2 * W), F32),
        mesh=plsc.VectorSubcoreMesh(**_MESH), scratch_types=scratch,
        compiler_params=_SC_PARAMS)
    def k(xm_hbm, ida_hbm, z_hbm, out_hbm, slab, idx2, vals, idx_t,
          val_t, semi, semv, semsc):
        cid = lax.axis_index("c")
        sid = lax.axis_index("s")
        col = cid * W
        zb = sid * (R // 16)
        pltpu.sync_copy(z_hbm.at[pl.ds(zb, R // 16)],
                        slab.at[pl.ds(zb, R // 16)])
        plsc.subcore_barrier()
        base = sid * per

        def group(g, carry):
            for b in range(NB):
                @pl.when(g > 0)
                def _(b=b):
                    pltpu.make_async_copy(
                        vals.at[pl.ds(b * CH, CH)],
                        slab.at[idx2.at[b]], semsc).wait()
            descs = []
            for b in range(NB):
                off = base + (g * NB + b) * CH
                descs.append(pltpu.async_copy(
                    ida_hbm.at[pl.ds(off, CH)], idx2.at[b], semi))
                descs.append(pltpu.async_copy(
                    xm_hbm.at[pl.ds(off, CH), pl.ds(col, W)],
                    vals.at[pl.ds(b * CH, CH)], semv))
            for d in descs:
                d.wait()
            for b in range(NB):
                pltpu.async_copy(vals.at[pl.ds(b * CH, CH)],
                                 slab.at[idx2.at[b]], semsc, add=True)
            return carry

        lax.fori_loop(0, NG, group, 0)
        for b in range(NB):
            pltpu.make_async_copy(vals.at[pl.ds(b * CH, CH)],
                                  slab.at[idx2.at[b]], semsc).wait()
        off = base + nfull * CH
        pltpu.sync_copy(ida_hbm.at[pl.ds(off, tail)], idx_t)
        pltpu.sync_copy(xm_hbm.at[pl.ds(off, tail), pl.ds(col, W)], val_t)
        pltpu.sync_copy(val_t, slab.at[idx_t], add=True)
        plsc.subcore_barrier()
        cb = sid * (A // 16)
        pltpu.sync_copy(slab.at[pl.ds(cb, A // 16)],
                        out_hbm.at[pl.ds(cb, A // 16), pl.ds(col, W)])

    return k(xm, id_a, zslab)


def kernel(h, m, rbf3, cbf3, Kidx3, id_swap, id3_expand_ba, id3_reduce_ca,
           rbf_h, id_c, id_a, W_dense_ca, W_dense_ba, W_mlp_rbf,
           W_bilinear, W_down, W_up_ca, W_up_ac, W_res_bef, W_res_aft,
           W_rbf_h, W_atom_dense1, W_atom_res, W_concat, W_res_m):
    Wb_t = jnp.transpose(W_bilinear, (1, 0, 2)).reshape(1024, 64)
    Wb_t = Wb_t.astype(jnp.bfloat16)
    i32 = lambda a: a.astype(I32)

    x_ba = _edge_pre(m, rbf3, W_dense_ba, W_mlp_rbf, W_down)
    x_ba_t = _sc_gather(x_ba, i32(id3_expand_ba))
    tmp = _bilinear(x_ba_t, cbf3, Wb_t)
    x = _sc_segsum_sorted(tmp, i32(id3_reduce_ca),
                          jnp.zeros((16016, 64), F32))
    x_sw = _sc_gather(x, i32(id_swap))
    m1, xm = _edge_mid(m, x, x_sw, rbf_h, W_dense_ca, W_up_ca, W_up_ac,
                       W_res_bef[0, 0], W_res_bef[0, 1],
                       W_res_aft[0, 0], W_res_aft[0, 1],
                       W_res_aft[1, 0], W_res_aft[1, 1], W_rbf_h)
    x2 = _sc_scatter_atoms(xm, i32(id_a), jnp.zeros((10016, 64), F32))
    h_new, g1, g2 = _atoms(x2, h, W_atom_dense1,
                           W_atom_res[0, 0], W_atom_res[0, 1],
                           W_atom_res[1, 0], W_atom_res[1, 1],
                           W_atom_res[2, 0], W_atom_res[2, 1],
                           W_concat[:128], W_concat[128:256])
    g1c, g2a = _sc_gather2(g1, i32(id_c), g2, i32(id_a))
    m_out = _edge_fin(g1c, g2a, m1, W_concat[256:], W_res_m[0, 0],
                      W_res_m[0, 1])
    return (h_new, m_out)


# speedup vs baseline: 1.2660x; 1.0419x over previous
import functools

import jax
import jax.numpy as jnp
from jax import lax
from jax.experimental import pallas as pl
from jax.experimental.pallas import tpu as pltpu
from jax.experimental.pallas import tpu_sc as plsc

F32 = jnp.float32
I32 = jnp.int32
INV_SQRT2 = 0.7071067811865475
ACT_SCALE = 1.0 / 0.6

N_ATOMS = 10000
N_EDGES = 160000
N_TRIP = 320000


def _act(x):
    return jax.nn.silu(x) * ACT_SCALE


def _res(x, w0, w1):
    y = _act(x @ w0[...])
    y = _act(y @ w1[...])
    return (x + y) * INV_SQRT2


def _edge_pre_kernel(m_ref, rbf3_ref, wba, wrbf, wdown, xba_ref):
    t = _act(m_ref[...] @ wba[...])
    t = t * (rbf3_ref[...] @ wrbf[...])
    xba_ref[...] = _act(t @ wdown[...])


def _edge_pre(m, rbf3, W_ba, W_rbf, W_down):
    E, BE = N_EDGES, 4000
    full = lambda s: pl.BlockSpec(s, lambda i: (0, 0))
    return pl.pallas_call(
        _edge_pre_kernel,
        grid=(E // BE,),
        in_specs=[pl.BlockSpec((BE, 128), lambda i: (i, 0)),
                  pl.BlockSpec((BE, 16), lambda i: (i, 0)),
                  full((128, 128)), full((16, 128)), full((128, 64))],
        out_specs=pl.BlockSpec((BE, 64), lambda i: (i, 0)),
        out_shape=jax.ShapeDtypeStruct((E, 64), F32),
    )(m, rbf3, W_ba, W_rbf, W_down)


def _bilinear_kernel(x_ref, c_ref, w_ref, out_ref, z_ref):
    x = x_ref[...].astype(jnp.bfloat16)
    c = c_ref[...].astype(jnp.bfloat16)
    z = jnp.concatenate([x * c[:, k:k + 1] for k in range(16)], axis=1)
    out_ref[...] = jnp.dot(z, w_ref[...], preferred_element_type=F32)


def _bilinear(x_ba_t, cbf3, Wb_t):
    T, BT = N_TRIP, 8000
    return pl.pallas_call(
        _bilinear_kernel,
        grid=(T // BT,),
        in_specs=[pl.BlockSpec((BT, 64), lambda i: (i, 0)),
                  pl.BlockSpec((BT, 16), lambda i: (i, 0)),
                  pl.BlockSpec((1024, 64), lambda i: (0, 0)),
                  ],
        out_specs=pl.BlockSpec((BT, 64), lambda i: (i, 0)),
        out_shape=jax.ShapeDtypeStruct((T, 64), F32),
        scratch_shapes=[pltpu.VMEM((BT, 1024), jnp.bfloat16)],
    )(x_ba_t, cbf3, Wb_t)


def _edge_mid_kernel(m_ref, x_ref, xsw_ref, rbfh_ref,
                     wca, wupca, wupac, bef0, bef1, a00, a01, a10, a11,
                     wrbfh, m1_ref, xm_ref):
    x_ca = _act(x_ref[...] @ wupca[...])
    x_ac = _act(xsw_ref[...] @ wupac[...])
    x3 = (x_ca + x_ac) * INV_SQRT2
    x4 = (_act(m_ref[...] @ wca[...]) + x3) * INV_SQRT2
    x4 = _res(x4, bef0, bef1)
    m1 = (m_ref[...] + x4) * INV_SQRT2
    m1 = _res(m1, a00, a01)
    m1 = _res(m1, a10, a11)
    m1_ref[...] = m1
    xm_ref[...] = m1 * (rbfh_ref[...] @ wrbfh[...])


def _edge_mid(m, x, x_sw, rbf_h, wca, wupca, wupac, bef0, bef1,
              a00, a01, a10, a11, wrbfh):
    E, BE = N_EDGES, 4000
    full = lambda s: pl.BlockSpec(s, lambda i: tuple(0 for _ in s))
    return pl.pallas_call(
        _edge_mid_kernel,
        grid=(E // BE,),
        in_specs=[pl.BlockSpec((BE, 128), lambda i: (i, 0)),
                  pl.BlockSpec((BE, 64), lambda i: (i, 0)),
                  pl.BlockSpec((BE, 64), lambda i: (i, 0)),
                  pl.BlockSpec((BE, 16), lambda i: (i, 0)),
                  full((128, 128)), full((64, 128)), full((64, 128)),
                  full((128, 128)), full((128, 128)), full((128, 128)),
                  full((128, 128)), full((128, 128)), full((128, 128)),
                  full((16, 128))],
        out_specs=[pl.BlockSpec((BE, 128), lambda i: (i, 0)),
                   pl.BlockSpec((BE, 128), lambda i: (i, 0))],
        out_shape=[jax.ShapeDtypeStruct((E, 128), F32),
                   jax.ShapeDtypeStruct((E, 128), F32)],
    )(m, x, x_sw, rbf_h, wca, wupca, wupac, bef0, bef1, a00, a01, a10,
      a11, wrbfh)


def _atom_kernel(x2_ref, h_ref, w1, r00, r01, r10, r11, r20, r21,
                 wc1, wc2, hout_ref, g1_ref, g2_ref):
    xh = _act(x2_ref[...] @ w1[...])
    xh = _res(xh, r00, r01)
    xh = _res(xh, r10, r11)
    xh = _res(xh, r20, r21)
    hn = (h_ref[...] + xh) * INV_SQRT2
    hout_ref[...] = hn
    g1_ref[...] = hn @ wc1[...]
    g2_ref[...] = hn @ wc2[...]


def _atoms(x2, h, w1, r00, r01, r10, r11, r20, r21, wc1, wc2):
    A, BA = N_ATOMS, 2000
    full = lambda s: pl.BlockSpec(s, lambda i: tuple(0 for _ in s))
    return pl.pallas_call(
        _atom_kernel,
        grid=(A // BA,),
        in_specs=[pl.BlockSpec((BA, 128), lambda i: (i, 0)),
                  pl.BlockSpec((BA, 128), lambda i: (i, 0)),
                  full((128, 128)), full((128, 128)), full((128, 128)),
                  full((128, 128)), full((128, 128)), full((128, 128)),
                  full((128, 128)), full((128, 128)), full((128, 128))],
        out_specs=[pl.BlockSpec((BA, 128), lambda i: (i, 0))] * 3,
        out_shape=[jax.ShapeDtypeStruct((A, 128), F32),
                   jax.ShapeDtypeStruct((A, 128), F32),
                   jax.ShapeDtypeStruct((A, 128), F32)],
    )(x2, h, w1, r00, r01, r10, r11, r20, r21, wc1, wc2)


def _edge_fin_kernel(g1_ref, g2_ref, m1_ref, wc3, rm0, rm1, out_ref):
    m1 = m1_ref[...]
    m2 = _act(g1_ref[...] + g2_ref[...] + m1 @ wc3[...])
    m2 = _res(m2, rm0, rm1)
    out_ref[...] = (m1 + m2) * INV_SQRT2


def _edge_fin(g1c, g2a, m1, wc3, rm0, rm1):
    E, BE = N_EDGES, 4000
    full = lambda s: pl.BlockSpec(s, lambda i: (0, 0))
    return pl.pallas_call(
        _edge_fin_kernel,
        grid=(E // BE,),
        in_specs=[pl.BlockSpec((BE, 128), lambda i: (i, 0)),
                  pl.BlockSpec((BE, 128), lambda i: (i, 0)),
                  pl.BlockSpec((BE, 128), lambda i: (i, 0)),
                  full((128, 128)), full((128, 128)), full((128, 128))],
        out_specs=pl.BlockSpec((BE, 128), lambda i: (i, 0)),
        out_shape=jax.ShapeDtypeStruct((E, 128), F32),
    )(g1c, g2a, m1, wc3, rm0, rm1)


_MESH = dict(core_axis_name="c", subcore_axis_name="s")
_SC_PARAMS = pltpu.CompilerParams(use_tc_tiling_on_sc=False,
                                  needs_layout_passes=False)


def _pick_nb(n):
    return next(nb for nb in (6, 4, 3, 2, 1) if n % nb == 0)


def _sc_gather(src, idx):
    N, W = src.shape
    dt = src.dtype
    B = idx.shape[0]
    per = B // 32
    CH = 128
    nfull = per // CH
    tail = per - nfull * CH
    NB = _pick_nb(nfull)
    NG = nfull // NB
    scratch = [pltpu.VMEM((per,), I32), pltpu.VMEM((NB * CH, W), dt),
               pltpu.SemaphoreType.DMA, pltpu.SemaphoreType.DMA]
    if tail:
        scratch += [pltpu.VMEM((tail, W), dt)]

    @functools.partial(
        pl.kernel, out_type=jax.ShapeDtypeStruct((B, W), dt),
        mesh=plsc.VectorSubcoreMesh(**_MESH), scratch_types=scratch,
        compiler_params=_SC_PARAMS)
    def k(src_hbm, idx_hbm, out_hbm, ids_v, rows_v, semg, semo, *rest):
        wid = lax.axis_index("s") * 2 + lax.axis_index("c")
        base = wid * per
        pltpu.sync_copy(idx_hbm.at[pl.ds(base, per)], ids_v)

        def group(g, carry):
            jo = g * NB * CH

            @pl.when(g > 0)
            def _():
                pltpu.make_async_copy(
                    rows_v, out_hbm.at[pl.ds(base, NB * CH)], semo).wait()

            descs = []
            for b in range(NB):
                descs.append(pltpu.async_copy(
                    src_hbm.at[ids_v.at[pl.ds(jo + b * CH, CH)]],
                    rows_v.at[pl.ds(b * CH, CH)], semg))
            for d in descs:
                d.wait()
            pltpu.async_copy(rows_v, out_hbm.at[pl.ds(base + jo, NB * CH)],
                             semo)
            return carry

        lax.fori_loop(0, NG, group, 0)
        if NG > 0:
            pltpu.make_async_copy(
                rows_v, out_hbm.at[pl.ds(base, NB * CH)], semo).wait()
        if tail:
            (rows_t,) = rest
            off = base + nfull * CH
            pltpu.async_copy(
                src_hbm.at[ids_v.at[pl.ds(nfull * CH, tail)]],
                rows_t, semg).wait()
            pltpu.sync_copy(rows_t, out_hbm.at[pl.ds(off, tail)])

    return k(src, idx)


def _sc_gather2(src1, idx_c, src2, idx_a):
    N, W = src1.shape
    dt = src1.dtype
    B = idx_c.shape[0]
    per = B // 32
    CH = 128
    nfull = per // CH
    tail = per - nfull * CH
    NB = _pick_nb(nfull)
    NG = nfull // NB
    out_t = [jax.ShapeDtypeStruct((B, W), dt)] * 2
    scratch = [pltpu.VMEM((per,), I32), pltpu.VMEM((per,), I32),
               pltpu.VMEM((NB * CH, W), dt), pltpu.VMEM((NB * CH, W), dt),
               pltpu.SemaphoreType.DMA, pltpu.SemaphoreType.DMA]
    if tail:
        scratch += [pltpu.VMEM((tail, W), dt), pltpu.VMEM((tail, W), dt)]

    @functools.partial(
        pl.kernel, out_type=out_t,
        mesh=plsc.VectorSubcoreMesh(**_MESH), scratch_types=scratch,
        compiler_params=_SC_PARAMS)
    def k(src_hbm, src2_hbm, idxc_hbm, idxa_hbm, outc_hbm, outa_hbm,
          idc_v, ida_v, rowsc_v, rowsa_v, semg, semo, *rest):
        wid = lax.axis_index("s") * 2 + lax.axis_index("c")
        base = wid * per
        pltpu.sync_copy(idxc_hbm.at[pl.ds(base, per)], idc_v)
        pltpu.sync_copy(idxa_hbm.at[pl.ds(base, per)], ida_v)

        def group(g, carry):
            jo = g * NB * CH

            @pl.when(g > 0)
            def _():
                pltpu.make_async_copy(
                    rowsc_v, outc_hbm.at[pl.ds(base, NB * CH)], semo).wait()
                pltpu.make_async_copy(
                    rowsa_v, outa_hbm.at[pl.ds(base, NB * CH)], semo).wait()

            descs = []
            for b in range(NB):
                descs.append(pltpu.async_copy(
                    src_hbm.at[idc_v.at[pl.ds(jo + b * CH, CH)]],
                    rowsc_v.at[pl.ds(b * CH, CH)], semg))
                descs.append(pltpu.async_copy(
                    src2_hbm.at[ida_v.at[pl.ds(jo + b * CH, CH)]],
                    rowsa_v.at[pl.ds(b * CH, CH)], semg))
            for d in descs:
                d.wait()
            pltpu.async_copy(rowsc_v, outc_hbm.at[pl.ds(base + jo, NB * CH)],
                             semo)
            pltpu.async_copy(rowsa_v, outa_hbm.at[pl.ds(base + jo, NB * CH)],
                             semo)
            return carry

        lax.fori_loop(0, NG, group, 0)
        if NG > 0:
            pltpu.make_async_copy(
                rowsc_v, outc_hbm.at[pl.ds(base, NB * CH)], semo).wait()
            pltpu.make_async_copy(
                rowsa_v, outa_hbm.at[pl.ds(base, NB * CH)], semo).wait()
        if tail:
            rows_tc, rows_ta = rest
            off = base + nfull * CH
            d1 = pltpu.async_copy(
                src_hbm.at[idc_v.at[pl.ds(nfull * CH, tail)]], rows_tc, semg)
            d2 = pltpu.async_copy(
                src2_hbm.at[ida_v.at[pl.ds(nfull * CH, tail)]], rows_ta,
                semg)
            d1.wait()
            d2.wait()
            pltpu.sync_copy(rows_tc, outc_hbm.at[pl.ds(off, tail)])
            pltpu.sync_copy(rows_ta, outa_hbm.at[pl.ds(off, tail)])

    return k(src1, src2, idx_c, idx_a)


def _sc_segsum_sorted(tmp, ids, zslab):
    T, E, W = N_TRIP, N_EDGES, 64
    RW = 16000
    NPASS = E // RW // 2
    CH = 128
    NCH = T // CH
    TPT = -(-NCH // 16)
    NB = 3
    NG = (TPT - 1) // NB
    R = 16016
    DUMP = 16000
    scratch = [pltpu.VMEM_SHARED((R, W), F32),
               pltpu.VMEM((TPT * CH,), I32),
               pltpu.VMEM((NB, CH), I32),
               pltpu.VMEM((NB * CH, W), F32),
               pltpu.SemaphoreType.DMA, pltpu.SemaphoreType.DMA,
               pltpu.SemaphoreType.DMA]

    @functools.partial(
        pl.kernel, out_type=jax.ShapeDtypeStruct((E, W), F32),
        mesh=plsc.VectorSubcoreMesh(**_MESH), scratch_types=scratch,
        compiler_params=_SC_PARAMS)
    def k(tmp_hbm, ids_hbm, z_hbm, out_hbm, slab, ids_v, idx2, vals,
          semp, semv, semsc):
        cid = lax.axis_index("c")
        sid = lax.axis_index("s")

        def src_row(t):
            return jnp.minimum(sid + 16 * t, NCH - 1) * CH

        def pre(t, carry):
            pltpu.async_copy(ids_hbm.at[pl.ds(src_row(t), CH)],
                             ids_v.at[pl.ds(t * CH, CH)], semp)

            @pl.when(t >= 8)
            def _():
                pltpu.make_async_copy(
                    ids_hbm.at[pl.ds(0, CH)], ids_v.at[pl.ds(0, CH)],
                    semp).wait()
            return carry

        lax.fori_loop(0, TPT, pre, 0)
        for _ in range(8):
            pltpu.make_async_copy(ids_hbm.at[pl.ds(0, CH)],
                                  ids_v.at[pl.ds(0, CH)], semp).wait()

        for p in range(NPASS):
            lo = (2 * p + cid) * RW
            zb = sid * (R // 16)
            pltpu.sync_copy(z_hbm.at[pl.ds(zb, R // 16)],
                            slab.at[pl.ds(zb, R // 16)])
            plsc.subcore_barrier()

            def active(t):
                minv = jnp.min(ids_v[pl.ds(t * CH, 16)])
                maxv = jnp.max(ids_v[pl.ds(t * CH + CH - 16, 16)])
                return ((maxv >= lo) & (minv < lo + RW)
                        & (sid + 16 * t < NCH))

            def fire(t, b):
                pltpu.async_copy(tmp_hbm.at[pl.ds(src_row(t), CH)],
                                 vals.at[pl.ds(b * CH, CH)], semv)

            def flush(t, b):
                pltpu.make_async_copy(tmp_hbm.at[pl.ds(src_row(t), CH)],
                                      vals.at[pl.ds(b * CH, CH)],
                                      semv).wait()
                row = idx2.at[b]
                for q in range(CH // 16):
                    iv = ids_v[pl.ds(t * CH + q * 16, 16)]
                    ok = (iv >= lo) & (iv < lo + RW)
                    row[pl.ds(q * 16, 16)] = jnp.where(ok, iv - lo, DUMP)
                pltpu.async_copy(vals.at[pl.ds(b * CH, CH)],
                                 slab.at[idx2.at[b]], semsc, add=True)

            def drain(b):
                pltpu.make_async_copy(vals.at[pl.ds(b * CH, CH)],
                                      slab.at[idx2.at[b]], semsc).wait()

            def group(g, carry):
                gp = jnp.maximum(g - 1, 0)
                for b in range(NB):
                    @pl.when((g > 0) & active(gp * NB + b))
                    def _(b=b):
                        drain(b)
                for b in range(NB):
                    t = g * NB + b

                    @pl.when(active(t))
                    def _(t=t, b=b):
                        fire(t, b)
                for b in range(NB):
                    t = g * NB + b

                    @pl.when(active(t))
                    def _(t=t, b=b):
                        flush(t, b)
                return carry

            lax.fori_loop(0, NG, group, 0)
            for b in range(NB):
                @pl.when(active((NG - 1) * NB + b))
                def _(b=b):
                    drain(b)
            t_ep = NG * NB
            for t in range(t_ep, TPT):
                @pl.when(active(t))
                def _(t=t):
                    fire(t, 0)
                    flush(t, 0)
                    drain(0)

            plsc.subcore_barrier()
            cb = sid * (RW // 16)
            pltpu.sync_copy(slab.at[pl.ds(cb, RW // 16)],
                            out_hbm.at[pl.ds(cb + lo, RW // 16)])
            plsc.subcore_barrier()

    return k(tmp, ids, zslab)


def _sc_scatter_atoms(xm, id_a, zslab):
    E, A, W = N_EDGES, N_ATOMS, 64
    per = E // 16
    CH = 128
    nfull = per // CH
    tail = per - nfull * CH
    NB = 6
    NG = nfull // NB
    R = 10016
    scratch = [pltpu.VMEM_SHARED((R, W), F32),
               pltpu.VMEM((NB, CH), I32), pltpu.VMEM((NB * CH, W), F32),
               pltpu.VMEM((tail,), I32), pltpu.VMEM((tail, W), F32),
               pltpu.SemaphoreType.DMA, pltpu.SemaphoreType.DMA,
               pltpu.SemaphoreType.DMA]

    @functools.partial(
        pl.kernel, out_type=jax.ShapeDtypeStruct((A, 2 * W), F32),
        mesh=plsc.VectorSubcoreMesh(**_MESH), scratch_types=scratch,
        compiler_params=_SC_PARAMS)
    def k(xm_hbm, ida_hbm, z_hbm, out_hbm, slab, idx2, vals, idx_t,
          val_t, semi, semv, semsc):
        cid = lax.axis_index("c")
        sid = lax.axis_index("s")
        col = cid * W
        zb = sid * (R // 16)
        pltpu.sync_copy(z_hbm.at[pl.ds(zb, R // 16)],
                        slab.at[pl.ds(zb, R // 16)])
        plsc.subcore_barrier()
        base = sid * per

        def group(g, carry):
            for b in range(NB):
                @pl.when(g > 0)
                def _(b=b):
                    pltpu.make_async_copy(
                        vals.at[pl.ds(b * CH, CH)],
                        slab.at[idx2.at[b]], semsc).wait()
            descs = []
            for b in range(NB):
                off = base + (g * NB + b) * CH
                descs.append(pltpu.async_copy(
                    ida_hbm.at[pl.ds(off, CH)], idx2.at[b], semi))
                descs.append(pltpu.async_copy(
                    xm_hbm.at[pl.ds(off, CH), pl.ds(col, W)],
                    vals.at[pl.ds(b * CH, CH)], semv))
            for d in descs:
                d.wait()
            for b in range(NB):
                pltpu.async_copy(vals.at[pl.ds(b * CH, CH)],
                                 slab.at[idx2.at[b]], semsc, add=True)
            return carry

        lax.fori_loop(0, NG, group, 0)
        for b in range(NB):
            pltpu.make_async_copy(vals.at[pl.ds(b * CH, CH)],
                                  slab.at[idx2.at[b]], semsc).wait()
        off = base + nfull * CH
        pltpu.sync_copy(ida_hbm.at[pl.ds(off, tail)], idx_t)
        pltpu.sync_copy(xm_hbm.at[pl.ds(off, tail), pl.ds(col, W)], val_t)
        pltpu.sync_copy(val_t, slab.at[idx_t], add=True)
        plsc.subcore_barrier()
        cb = sid * (A // 16)
        pltpu.sync_copy(slab.at[pl.ds(cb, A // 16)],
                        out_hbm.at[pl.ds(cb, A // 16), pl.ds(col, W)])

    return k(xm, id_a, zslab)


def kernel(h, m, rbf3, cbf3, Kidx3, id_swap, id3_expand_ba, id3_reduce_ca,
           rbf_h, id_c, id_a, W_dense_ca, W_dense_ba, W_mlp_rbf,
           W_bilinear, W_down, W_up_ca, W_up_ac, W_res_bef, W_res_aft,
           W_rbf_h, W_atom_dense1, W_atom_res, W_concat, W_res_m):
    Wb_t = jnp.transpose(W_bilinear, (1, 0, 2)).reshape(1024, 64)
    Wb_t = Wb_t.astype(jnp.bfloat16)
    i32 = lambda a: a.astype(I32)

    x_ba = _edge_pre(m, rbf3, W_dense_ba, W_mlp_rbf, W_down)
    x_ba_t = _sc_gather(x_ba, i32(id3_expand_ba))
    tmp = _bilinear(x_ba_t, cbf3, Wb_t)
    x = _sc_segsum_sorted(tmp, i32(id3_reduce_ca),
                          jnp.zeros((16016, 64), F32))
    x_sw = _sc_gather(x, i32(id_swap))
    m1, xm = _edge_mid(m, x, x_sw, rbf_h, W_dense_ca, W_up_ca, W_up_ac,
                       W_res_bef[0, 0], W_res_bef[0, 1],
                       W_res_aft[0, 0], W_res_aft[0, 1],
                       W_res_aft[1, 0], W_res_aft[1, 1], W_rbf_h)
    x2 = _sc_scatter_atoms(xm, i32(id_a), jnp.zeros((10016, 64), F32))
    h_new, g1, g2 = _atoms(x2, h, W_atom_dense1,
                           W_atom_res[0, 0], W_atom_res[0, 1],
                           W_atom_res[1, 0], W_atom_res[1, 1],
                           W_atom_res[2, 0], W_atom_res[2, 1],
                           W_concat[:128], W_concat[128:256])
    g1c, g2a = _sc_gather2(g1, i32(id_c), g2, i32(id_a))
    m_out = _edge_fin(g1c, g2a, m1, W_concat[256:], W_res_m[0, 0],
                      W_res_m[0, 1])
    return (h_new, m_out)


# speedup vs baseline: 1.2744x; 1.0066x over previous
import functools

import jax
import jax.numpy as jnp
from jax import lax
from jax.experimental import pallas as pl
from jax.experimental.pallas import tpu as pltpu
from jax.experimental.pallas import tpu_sc as plsc

F32 = jnp.float32
I32 = jnp.int32
INV_SQRT2 = 0.7071067811865475
ACT_SCALE = 1.0 / 0.6

N_ATOMS = 10000
N_EDGES = 160000
N_TRIP = 320000


def _act(x):
    return jax.nn.silu(x) * ACT_SCALE


def _res(x, w0, w1):
    y = _act(x @ w0[...])
    y = _act(y @ w1[...])
    return (x + y) * INV_SQRT2


def _edge_pre_kernel(m_ref, rbf3_ref, wba, wrbf, wdown, xba_ref):
    t = _act(m_ref[...] @ wba[...])
    t = t * (rbf3_ref[...] @ wrbf[...])
    xba_ref[...] = _act(t @ wdown[...])


def _edge_pre(m, rbf3, W_ba, W_rbf, W_down):
    E, BE = N_EDGES, 8000
    full = lambda s: pl.BlockSpec(s, lambda i: (0, 0))
    return pl.pallas_call(
        _edge_pre_kernel,
        grid=(E // BE,),
        in_specs=[pl.BlockSpec((BE, 128), lambda i: (i, 0)),
                  pl.BlockSpec((BE, 16), lambda i: (i, 0)),
                  full((128, 128)), full((16, 128)), full((128, 64))],
        out_specs=pl.BlockSpec((BE, 64), lambda i: (i, 0)),
        out_shape=jax.ShapeDtypeStruct((E, 64), F32),
    )(m, rbf3, W_ba, W_rbf, W_down)


def _bilinear_kernel(x_ref, c_ref, w_ref, out_ref, z_ref):
    x = x_ref[...].astype(jnp.bfloat16)
    c = c_ref[...].astype(jnp.bfloat16)
    z = jnp.concatenate([x * c[:, k:k + 1] for k in range(16)], axis=1)
    out_ref[...] = jnp.dot(z, w_ref[...], preferred_element_type=F32)


def _bilinear(x_ba_t, cbf3, Wb_t):
    T, BT = N_TRIP, 8000
    return pl.pallas_call(
        _bilinear_kernel,
        grid=(T // BT,),
        in_specs=[pl.BlockSpec((BT, 64), lambda i: (i, 0)),
                  pl.BlockSpec((BT, 16), lambda i: (i, 0)),
                  pl.BlockSpec((1024, 64), lambda i: (0, 0)),
                  ],
        out_specs=pl.BlockSpec((BT, 64), lambda i: (i, 0)),
        out_shape=jax.ShapeDtypeStruct((T, 64), F32),
        scratch_shapes=[pltpu.VMEM((BT, 1024), jnp.bfloat16)],
    )(x_ba_t, cbf3, Wb_t)


def _edge_mid_kernel(m_ref, x_ref, xsw_ref, rbfh_ref,
                     wca, wupca, wupac, bef0, bef1, a00, a01, a10, a11,
                     wrbfh, m1_ref, xm_ref):
    x_ca = _act(x_ref[...] @ wupca[...])
    x_ac = _act(xsw_ref[...] @ wupac[...])
    x3 = (x_ca + x_ac) * INV_SQRT2
    x4 = (_act(m_ref[...] @ wca[...]) + x3) * INV_SQRT2
    x4 = _res(x4, bef0, bef1)
    m1 = (m_ref[...] + x4) * INV_SQRT2
    m1 = _res(m1, a00, a01)
    m1 = _res(m1, a10, a11)
    m1_ref[...] = m1
    xm_ref[...] = m1 * (rbfh_ref[...] @ wrbfh[...])


def _edge_mid(m, x, x_sw, rbf_h, wca, wupca, wupac, bef0, bef1,
              a00, a01, a10, a11, wrbfh):
    E, BE = N_EDGES, 8000
    full = lambda s: pl.BlockSpec(s, lambda i: tuple(0 for _ in s))
    return pl.pallas_call(
        _edge_mid_kernel,
        grid=(E // BE,),
        in_specs=[pl.BlockSpec((BE, 128), lambda i: (i, 0)),
                  pl.BlockSpec((BE, 64), lambda i: (i, 0)),
                  pl.BlockSpec((BE, 64), lambda i: (i, 0)),
                  pl.BlockSpec((BE, 16), lambda i: (i, 0)),
                  full((128, 128)), full((64, 128)), full((64, 128)),
                  full((128, 128)), full((128, 128)), full((128, 128)),
                  full((128, 128)), full((128, 128)), full((128, 128)),
                  full((16, 128))],
        out_specs=[pl.BlockSpec((BE, 128), lambda i: (i, 0)),
                   pl.BlockSpec((BE, 128), lambda i: (i, 0))],
        out_shape=[jax.ShapeDtypeStruct((E, 128), F32),
                   jax.ShapeDtypeStruct((E, 128), F32)],
    )(m, x, x_sw, rbf_h, wca, wupca, wupac, bef0, bef1, a00, a01, a10,
      a11, wrbfh)


def _atom_kernel(x2_ref, h_ref, w1, r00, r01, r10, r11, r20, r21,
                 wc1, wc2, hout_ref, g1_ref, g2_ref):
    xh = _act(x2_ref[...] @ w1[...])
    xh = _res(xh, r00, r01)
    xh = _res(xh, r10, r11)
    xh = _res(xh, r20, r21)
    hn = (h_ref[...] + xh) * INV_SQRT2
    hout_ref[...] = hn
    g1_ref[...] = hn @ wc1[...]
    g2_ref[...] = hn @ wc2[...]


def _atoms(x2, h, w1, r00, r01, r10, r11, r20, r21, wc1, wc2):
    A, BA = N_ATOMS, 2000
    full = lambda s: pl.BlockSpec(s, lambda i: tuple(0 for _ in s))
    return pl.pallas_call(
        _atom_kernel,
        grid=(A // BA,),
        in_specs=[pl.BlockSpec((BA, 128), lambda i: (i, 0)),
                  pl.BlockSpec((BA, 128), lambda i: (i, 0)),
                  full((128, 128)), full((128, 128)), full((128, 128)),
                  full((128, 128)), full((128, 128)), full((128, 128)),
                  full((128, 128)), full((128, 128)), full((128, 128))],
        out_specs=[pl.BlockSpec((BA, 128), lambda i: (i, 0))] * 3,
        out_shape=[jax.ShapeDtypeStruct((A, 128), F32),
                   jax.ShapeDtypeStruct((A, 128), F32),
                   jax.ShapeDtypeStruct((A, 128), F32)],
    )(x2, h, w1, r00, r01, r10, r11, r20, r21, wc1, wc2)


def _edge_fin_kernel(g1_ref, g2_ref, m1_ref, wc3, rm0, rm1, out_ref):
    m1 = m1_ref[...]
    m2 = _act(g1_ref[...] + g2_ref[...] + m1 @ wc3[...])
    m2 = _res(m2, rm0, rm1)
    out_ref[...] = (m1 + m2) * INV_SQRT2


def _edge_fin(g1c, g2a, m1, wc3, rm0, rm1):
    E, BE = N_EDGES, 8000
    full = lambda s: pl.BlockSpec(s, lambda i: (0, 0))
    return pl.pallas_call(
        _edge_fin_kernel,
        grid=(E // BE,),
        in_specs=[pl.BlockSpec((BE, 128), lambda i: (i, 0)),
                  pl.BlockSpec((BE, 128), lambda i: (i, 0)),
                  pl.BlockSpec((BE, 128), lambda i: (i, 0)),
                  full((128, 128)), full((128, 128)), full((128, 128))],
        out_specs=pl.BlockSpec((BE, 128), lambda i: (i, 0)),
        out_shape=jax.ShapeDtypeStruct((E, 128), F32),
    )(g1c, g2a, m1, wc3, rm0, rm1)


_MESH = dict(core_axis_name="c", subcore_axis_name="s")
_SC_PARAMS = pltpu.CompilerParams(use_tc_tiling_on_sc=False,
                                  needs_layout_passes=False)


def _pick_nb(n):
    return next(nb for nb in (6, 4, 3, 2, 1) if n % nb == 0)


def _sc_gather(src, idx):
    N, W = src.shape
    dt = src.dtype
    B = idx.shape[0]
    per = B // 32
    CH = 128
    nfull = per // CH
    tail = per - nfull * CH
    NB = _pick_nb(nfull)
    NG = nfull // NB
    scratch = [pltpu.VMEM((per,), I32), pltpu.VMEM((NB * CH, W), dt),
               pltpu.SemaphoreType.DMA, pltpu.SemaphoreType.DMA]
    if tail:
        scratch += [pltpu.VMEM((tail, W), dt)]

    @functools.partial(
        pl.kernel, out_type=jax.ShapeDtypeStruct((B, W), dt),
        mesh=plsc.VectorSubcoreMesh(**_MESH), scratch_types=scratch,
        compiler_params=_SC_PARAMS)
    def k(src_hbm, idx_hbm, out_hbm, ids_v, rows_v, semg, semo, *rest):
        wid = lax.axis_index("s") * 2 + lax.axis_index("c")
        base = wid * per
        pltpu.sync_copy(idx_hbm.at[pl.ds(base, per)], ids_v)

        def group(g, carry):
            jo = g * NB * CH

            @pl.when(g > 0)
            def _():
                pltpu.make_async_copy(
                    rows_v, out_hbm.at[pl.ds(base, NB * CH)], semo).wait()

            descs = []
            for b in range(NB):
                descs.append(pltpu.async_copy(
                    src_hbm.at[ids_v.at[pl.ds(jo + b * CH, CH)]],
                    rows_v.at[pl.ds(b * CH, CH)], semg))
            for d in descs:
                d.wait()
            pltpu.async_copy(rows_v, out_hbm.at[pl.ds(base + jo, NB * CH)],
                             semo)
            return carry

        lax.fori_loop(0, NG, group, 0)
        if NG > 0:
            pltpu.make_async_copy(
                rows_v, out_hbm.at[pl.ds(base, NB * CH)], semo).wait()
        if tail:
            (rows_t,) = rest
            off = base + nfull * CH
            pltpu.async_copy(
                src_hbm.at[ids_v.at[pl.ds(nfull * CH, tail)]],
                rows_t, semg).wait()
            pltpu.sync_copy(rows_t, out_hbm.at[pl.ds(off, tail)])

    return k(src, idx)


def _sc_gather2(src1, idx_c, src2, idx_a):
    N, W = src1.shape
    dt = src1.dtype
    B = idx_c.shape[0]
    per = B // 32
    CH = 128
    nfull = per // CH
    tail = per - nfull * CH
    NB = _pick_nb(nfull)
    NG = nfull // NB
    out_t = [jax.ShapeDtypeStruct((B, W), dt)] * 2
    scratch = [pltpu.VMEM((per,), I32), pltpu.VMEM((per,), I32),
               pltpu.VMEM((NB * CH, W), dt), pltpu.VMEM((NB * CH, W), dt),
               pltpu.SemaphoreType.DMA, pltpu.SemaphoreType.DMA]
    if tail:
        scratch += [pltpu.VMEM((tail, W), dt), pltpu.VMEM((tail, W), dt)]

    @functools.partial(
        pl.kernel, out_type=out_t,
        mesh=plsc.VectorSubcoreMesh(**_MESH), scratch_types=scratch,
        compiler_params=_SC_PARAMS)
    def k(src_hbm, src2_hbm, idxc_hbm, idxa_hbm, outc_hbm, outa_hbm,
          idc_v, ida_v, rowsc_v, rowsa_v, semg, semo, *rest):
        wid = lax.axis_index("s") * 2 + lax.axis_index("c")
        base = wid * per
        pltpu.sync_copy(idxc_hbm.at[pl.ds(base, per)], idc_v)
        pltpu.sync_copy(idxa_hbm.at[pl.ds(base, per)], ida_v)

        def group(g, carry):
            jo = g * NB * CH

            @pl.when(g > 0)
            def _():
                pltpu.make_async_copy(
                    rowsc_v, outc_hbm.at[pl.ds(base, NB * CH)], semo).wait()
                pltpu.make_async_copy(
                    rowsa_v, outa_hbm.at[pl.ds(base, NB * CH)], semo).wait()

            descs = []
            for b in range(NB):
                descs.append(pltpu.async_copy(
                    src_hbm.at[idc_v.at[pl.ds(jo + b * CH, CH)]],
                    rowsc_v.at[pl.ds(b * CH, CH)], semg))
                descs.append(pltpu.async_copy(
                    src2_hbm.at[ida_v.at[pl.ds(jo + b * CH, CH)]],
                    rowsa_v.at[pl.ds(b * CH, CH)], semg))
            for d in descs:
                d.wait()
            pltpu.async_copy(rowsc_v, outc_hbm.at[pl.ds(base + jo, NB * CH)],
                             semo)
            pltpu.async_copy(rowsa_v, outa_hbm.at[pl.ds(base + jo, NB * CH)],
                             semo)
            return carry

        lax.fori_loop(0, NG, group, 0)
        if NG > 0:
            pltpu.make_async_copy(
                rowsc_v, outc_hbm.at[pl.ds(base, NB * CH)], semo).wait()
            pltpu.make_async_copy(
                rowsa_v, outa_hbm.at[pl.ds(base, NB * CH)], semo).wait()
        if tail:
            rows_tc, rows_ta = rest
            off = base + nfull * CH
            d1 = pltpu.async_copy(
                src_hbm.at[idc_v.at[pl.ds(nfull * CH, tail)]], rows_tc, semg)
            d2 = pltpu.async_copy(
                src2_hbm.at[ida_v.at[pl.ds(nfull * CH, tail)]], rows_ta,
                semg)
            d1.wait()
            d2.wait()
            pltpu.sync_copy(rows_tc, outc_hbm.at[pl.ds(off, tail)])
            pltpu.sync_copy(rows_ta, outa_hbm.at[pl.ds(off, tail)])

    return k(src1, src2, idx_c, idx_a)


def _sc_segsum_sorted(tmp, ids, zslab):
    T, E, W = N_TRIP, N_EDGES, 64
    RW = 16000
    NPASS = E // RW // 2
    CH = 128
    NCH = T // CH
    TPT = -(-NCH // 16)
    NB = 3
    NG = (TPT - 1) // NB
    R = 16016
    DUMP = 16000
    scratch = [pltpu.VMEM_SHARED((R, W), F32),
               pltpu.VMEM((TPT * CH,), I32),
               pltpu.VMEM((NB, CH), I32),
               pltpu.VMEM((NB * CH, W), F32),
               pltpu.SemaphoreType.DMA, pltpu.SemaphoreType.DMA,
               pltpu.SemaphoreType.DMA]

    @functools.partial(
        pl.kernel, out_type=jax.ShapeDtypeStruct((E, W), F32),
        mesh=plsc.VectorSubcoreMesh(**_MESH), scratch_types=scratch,
        compiler_params=_SC_PARAMS)
    def k(tmp_hbm, ids_hbm, z_hbm, out_hbm, slab, ids_v, idx2, vals,
          semp, semv, semsc):
        cid = lax.axis_index("c")
        sid = lax.axis_index("s")

        def src_row(t):
            return jnp.minimum(sid + 16 * t, NCH - 1) * CH

        def pre(t, carry):
            pltpu.async_copy(ids_hbm.at[pl.ds(src_row(t), CH)],
                             ids_v.at[pl.ds(t * CH, CH)], semp)

            @pl.when(t >= 8)
            def _():
                pltpu.make_async_copy(
                    ids_hbm.at[pl.ds(0, CH)], ids_v.at[pl.ds(0, CH)],
                    semp).wait()
            return carry

        lax.fori_loop(0, TPT, pre, 0)
        for _ in range(8):
            pltpu.make_async_copy(ids_hbm.at[pl.ds(0, CH)],
                                  ids_v.at[pl.ds(0, CH)], semp).wait()

        for p in range(NPASS):
            lo = (2 * p + cid) * RW
            zb = sid * (R // 16)
            pltpu.sync_copy(z_hbm.at[pl.ds(zb, R // 16)],
                            slab.at[pl.ds(zb, R // 16)])
            plsc.subcore_barrier()

            def active(t):
                minv = jnp.min(ids_v[pl.ds(t * CH, 16)])
                maxv = jnp.max(ids_v[pl.ds(t * CH + CH - 16, 16)])
                return ((maxv >= lo) & (minv < lo + RW)
                        & (sid + 16 * t < NCH))

            def fire(t, b):
                pltpu.async_copy(tmp_hbm.at[pl.ds(src_row(t), CH)],
                                 vals.at[pl.ds(b * CH, CH)], semv)

            def flush(t, b):
                pltpu.make_async_copy(tmp_hbm.at[pl.ds(src_row(t), CH)],
                                      vals.at[pl.ds(b * CH, CH)],
                                      semv).wait()
                row = idx2.at[b]
                for q in range(CH // 16):
                    iv = ids_v[pl.ds(t * CH + q * 16, 16)]
                    ok = (iv >= lo) & (iv < lo + RW)
                    row[pl.ds(q * 16, 16)] = jnp.where(ok, iv - lo, DUMP)
                pltpu.async_copy(vals.at[pl.ds(b * CH, CH)],
                                 slab.at[idx2.at[b]], semsc, add=True)

            def drain(b):
                pltpu.make_async_copy(vals.at[pl.ds(b * CH, CH)],
                                      slab.at[idx2.at[b]], semsc).wait()

            def group(g, carry):
                gp = jnp.maximum(g - 1, 0)
                for b in range(NB):
                    @pl.when((g > 0) & active(gp * NB + b))
                    def _(b=b):
                        drain(b)
                for b in range(NB):
                    t = g * NB + b

                    @pl.when(active(t))
                    def _(t=t, b=b):
                        fire(t, b)
                for b in range(NB):
                    t = g * NB + b

                    @pl.when(active(t))
                    def _(t=t, b=b):
                        flush(t, b)
                return carry

            lax.fori_loop(0, NG, group, 0)
            for b in range(NB):
                @pl.when(active((NG - 1) * NB + b))
                def _(b=b):
                    drain(b)
            t_ep = NG * NB
            for t in range(t_ep, TPT):
                @pl.when(active(t))
                def _(t=t):
                    fire(t, 0)
                    flush(t, 0)
                    drain(0)

            plsc.subcore_barrier()
            cb = sid * (RW // 16)
            pltpu.sync_copy(slab.at[pl.ds(cb, RW // 16)],
                            out_hbm.at[pl.ds(cb + lo, RW // 16)])
            plsc.subcore_barrier()

    return k(tmp, ids, zslab)


def _sc_scatter_atoms(xm, id_a, zslab):
    E, A, W = N_EDGES, N_ATOMS, 64
    per = E // 16
    CH = 128
    nfull = per // CH
    tail = per - nfull * CH
    NB = 6
    NG = nfull // NB
    R = 10016
    scratch = [pltpu.VMEM_SHARED((R, W), F32),
               pltpu.VMEM((NB, CH), I32), pltpu.VMEM((NB * CH, W), F32),
               pltpu.VMEM((tail,), I32), pltpu.VMEM((tail, W), F32),
               pltpu.SemaphoreType.DMA, pltpu.SemaphoreType.DMA,
               pltpu.SemaphoreType.DMA]

    @functools.partial(
        pl.kernel, out_type=jax.ShapeDtypeStruct((A, 2 * W), F32),
        mesh=plsc.VectorSubcoreMesh(**_MESH), scratch_types=scratch,
        compiler_params=_SC_PARAMS)
    def k(xm_hbm, ida_hbm, z_hbm, out_hbm, slab, idx2, vals, idx_t,
          val_t, semi, semv, semsc):
        cid = lax.axis_index("c")
        sid = lax.axis_index("s")
        col = cid * W
        zb = sid * (R // 16)
        pltpu.sync_copy(z_hbm.at[pl.ds(zb, R // 16)],
                        slab.at[pl.ds(zb, R // 16)])
        plsc.subcore_barrier()
        base = sid * per

        def group(g, carry):
            for b in range(NB):
                @pl.when(g > 0)
                def _(b=b):
                    pltpu.make_async_copy(
                        vals.at[pl.ds(b * CH, CH)],
                        slab.at[idx2.at[b]], semsc).wait()
            descs = []
            for b in range(NB):
                off = base + (g * NB + b) * CH
                descs.append(pltpu.async_copy(
                    ida_hbm.at[pl.ds(off, CH)], idx2.at[b], semi))
                descs.append(pltpu.async_copy(
                    xm_hbm.at[pl.ds(off, CH), pl.ds(col, W)],
                    vals.at[pl.ds(b * CH, CH)], semv))
            for d in descs:
                d.wait()
            for b in range(NB):
                pltpu.async_copy(vals.at[pl.ds(b * CH, CH)],
                                 slab.at[idx2.at[b]], semsc, add=True)
            return carry

        lax.fori_loop(0, NG, group, 0)
        for b in range(NB):
            pltpu.make_async_copy(vals.at[pl.ds(b * CH, CH)],
                                  slab.at[idx2.at[b]], semsc).wait()
        off = base + nfull * CH
        pltpu.sync_copy(ida_hbm.at[pl.ds(off, tail)], idx_t)
        pltpu.sync_copy(xm_hbm.at[pl.ds(off, tail), pl.ds(col, W)], val_t)
        pltpu.sync_copy(val_t, slab.at[idx_t], add=True)
        plsc.subcore_barrier()
        cb = sid * (A // 16)
        pltpu.sync_copy(slab.at[pl.ds(cb, A // 16)],
                        out_hbm.at[pl.ds(cb, A // 16), pl.ds(col, W)])

    return k(xm, id_a, zslab)


def kernel(h, m, rbf3, cbf3, Kidx3, id_swap, id3_expand_ba, id3_reduce_ca,
           rbf_h, id_c, id_a, W_dense_ca, W_dense_ba, W_mlp_rbf,
           W_bilinear, W_down, W_up_ca, W_up_ac, W_res_bef, W_res_aft,
           W_rbf_h, W_atom_dense1, W_atom_res, W_concat, W_res_m):
    Wb_t = jnp.transpose(W_bilinear, (1, 0, 2)).reshape(1024, 64)
    Wb_t = Wb_t.astype(jnp.bfloat16)
    i32 = lambda a: a.astype(I32)

    x_ba = _edge_pre(m, rbf3, W_dense_ba, W_mlp_rbf, W_down)
    x_ba_t = _sc_gather(x_ba, i32(id3_expand_ba))
    tmp = _bilinear(x_ba_t, cbf3, Wb_t)
    x = _sc_segsum_sorted(tmp, i32(id3_reduce_ca),
                          jnp.zeros((16016, 64), F32))
    x_sw = _sc_gather(x, i32(id_swap))
    m1, xm = _edge_mid(m, x, x_sw, rbf_h, W_dense_ca, W_up_ca, W_up_ac,
                       W_res_bef[0, 0], W_res_bef[0, 1],
                       W_res_aft[0, 0], W_res_aft[0, 1],
                       W_res_aft[1, 0], W_res_aft[1, 1], W_rbf_h)
    x2 = _sc_scatter_atoms(xm, i32(id_a), jnp.zeros((10016, 64), F32))
    h_new, g1, g2 = _atoms(x2, h, W_atom_dense1,
                           W_atom_res[0, 0], W_atom_res[0, 1],
                           W_atom_res[1, 0], W_atom_res[1, 1],
                           W_atom_res[2, 0], W_atom_res[2, 1],
                           W_concat[:128], W_concat[128:256])
    g1c, g2a = _sc_gather2(g1, i32(id_c), g2, i32(id_a))
    m_out = _edge_fin(g1c, g2a, m1, W_concat[256:], W_res_m[0, 0],
                      W_res_m[0, 1])
    return (h_new, m_out)
